# baseline (speedup 1.0000x reference)
import functools

import numpy as np
import jax
import jax.numpy as jnp
from jax import lax
from jax.experimental import pallas as pl
from jax.experimental.pallas import tpu as pltpu
from jax.experimental.pallas import tpu_sc as plsc

N = 10080
E = 40320
DIN = 2048
C1 = 1024
C2 = 64

NPAD = 10240
EPAD = 40448
SENT = 10200
R = 512
RS = R + 128
CPS = 10
STRIPE = R // 16

_MESH = dict(core_axis_name="c", subcore_axis_name="s")


def _z0():
    return jnp.int32(0)


def _mm_kernel(a_ref, w_ref, b_ref, o_ref):
    o_ref[...] = (jnp.dot(a_ref[...], w_ref[...],
                          preferred_element_type=jnp.float32) + b_ref[...])


def _mm(a, w, b, bm, bn):
    m, k = a.shape
    _, n = w.shape
    return pl.pallas_call(
        _mm_kernel,
        grid=(m // bm, n // bn),
        in_specs=[pl.BlockSpec((bm, k), lambda i, j: (i, _z0())),
                  pl.BlockSpec((k, bn), lambda i, j: (_z0(), j)),
                  pl.BlockSpec((1, bn), lambda i, j: (_z0(), j))],
        out_specs=pl.BlockSpec((bm, bn), lambda i, j: (i, j)),
        out_shape=jax.ShapeDtypeStruct((m, n), jnp.float32),
    )(a, w, b.reshape(1, n))


def _fuse_mm_kernel(nt_ref, dt_ref, s1_ref, ng_ref, dg_ref,
                    wa_ref, wb_ref, b_ref, o_ref):
    ns = nt_ref.shape[0]
    sw = nt_ref.shape[2]
    dt = jnp.maximum(dt_ref[...], 1e-30)
    dg = jnp.maximum(dg_ref[...], 1e-30)
    acc = jnp.zeros(o_ref.shape, jnp.float32)
    for s in range(ns):
        h1s = jnp.maximum(nt_ref[s] / dt + s1_ref[:, s * sw:(s + 1) * sw], 0.0)
        z1s = jnp.maximum(ng_ref[s] / dg, 0.0)
        acc = acc + jnp.dot(h1s, wa_ref[s], preferred_element_type=jnp.float32)
        acc = acc + jnp.dot(z1s, wb_ref[s], preferred_element_type=jnp.float32)
    o_ref[...] = acc + b_ref[...]


def _fuse_mm(nt, dt, s1, ng, dg, wa, wb, b, bm):
    ns, m, sw = nt.shape
    n = wa.shape[2]
    return pl.pallas_call(
        _fuse_mm_kernel,
        grid=(m // bm,),
        in_specs=[pl.BlockSpec((ns, bm, sw), lambda i: (_z0(), i, _z0())),
                  pl.BlockSpec((bm, 1), lambda i: (i, _z0())),
                  pl.BlockSpec((bm, ns * sw), lambda i: (i, _z0())),
                  pl.BlockSpec((ns, bm, sw), lambda i: (_z0(), i, _z0())),
                  pl.BlockSpec((bm, 1), lambda i: (i, _z0())),
                  pl.BlockSpec((ns, sw, n), lambda i: (_z0(), _z0(), _z0())),
                  pl.BlockSpec((ns, sw, n), lambda i: (_z0(), _z0(), _z0())),
                  pl.BlockSpec((1, n), lambda i: (_z0(), _z0()))],
        out_specs=pl.BlockSpec((bm, n), lambda i: (i, _z0())),
        out_shape=jax.ShapeDtypeStruct((m, n), jnp.float32),
    )(nt, dt, s1, ng, dg, wa, wb, b.reshape(1, n))


def _fuse_final_kernel(nt_ref, dt_ref, s2_ref, ng_ref, dg_ref, o_ref):
    kw = s2_ref.shape[1]
    h2 = jnp.maximum(nt_ref[0][:, :kw] / jnp.maximum(dt_ref[...], 1e-30)
                     + s2_ref[...], 0.0)
    z2 = jnp.maximum(ng_ref[0][:, :kw] / jnp.maximum(dg_ref[...], 1e-30), 0.0)
    o_ref[...] = h2 + z2


def _fuse_final(nt, dt, s2, ng, dg, bm):
    _, m, w = nt.shape
    k = s2.shape[1]
    return pl.pallas_call(
        _fuse_final_kernel,
        grid=(m // bm,),
        in_specs=[pl.BlockSpec((1, bm, w), lambda i: (_z0(), i, _z0())),
                  pl.BlockSpec((bm, 1), lambda i: (i, _z0())),
                  pl.BlockSpec((bm, k), lambda i: (i, _z0())),
                  pl.BlockSpec((1, bm, w), lambda i: (_z0(), i, _z0())),
                  pl.BlockSpec((bm, 1), lambda i: (i, _z0()))],
        out_specs=pl.BlockSpec((bm, k), lambda i: (i, _z0())),
        out_shape=jax.ShapeDtypeStruct((m, k), jnp.float32),
    )(nt, dt, s2, ng, dg)


def _mlp_kernel(g_ref, w1_ref, b1_ref, w2_ref, b2_ref, w3_ref, b3_ref,
                w4_ref, b4_ref, o_ref):
    f32 = jnp.float32
    g = jnp.maximum(jnp.dot(g_ref[...], w1_ref[...],
                            preferred_element_type=f32) + b1_ref[...], 0.0)
    g = jnp.maximum(jnp.dot(g, w2_ref[...],
                            preferred_element_type=f32) + b2_ref[...], 0.0)
    g = jnp.maximum(jnp.dot(g, w3_ref[...],
                            preferred_element_type=f32) + b3_ref[...], 0.0)
    o_ref[...] = jnp.dot(g, w4_ref[...],
                         preferred_element_type=f32) + b4_ref[...]


def _mlp(g, w1, b1, w2, b2, w3, b3, w4, b4):
    return pl.pallas_call(
        _mlp_kernel,
        out_shape=jax.ShapeDtypeStruct((g.shape[0], w4.shape[1]), jnp.float32),
    )(g, w1, b1.reshape(1, -1), w2, b2.reshape(1, -1),
      w3, b3.reshape(1, -1), w4, b4.reshape(1, -1))


def _make_sc_alpha(C, W):
    ET = EPAD // 32
    NB = ET // 16
    scale = 1.0 / float(np.sqrt(C))

    @functools.partial(
        pl.kernel,
        out_type=jax.ShapeDtypeStruct((EPAD,), jnp.float32),
        mesh=plsc.VectorSubcoreMesh(**_MESH),
        compiler_params=pltpu.CompilerParams(needs_layout_passes=False),
        scratch_types=[
            pltpu.VMEM((ET,), jnp.int32),
            pltpu.VMEM((ET,), jnp.int32),
            pltpu.VMEM((ET,), jnp.float32),
            pltpu.VMEM((16, W), jnp.float32),
            pltpu.VMEM((16, W), jnp.float32),
            pltpu.VMEM((16, W), jnp.float32),
            pltpu.VMEM((16, W), jnp.float32),
            pltpu.SemaphoreType.DMA,
            pltpu.SemaphoreType.DMA,
            pltpu.SemaphoreType.DMA,
            pltpu.SemaphoreType.DMA,
        ],
    )
    def k(q_hbm, k_hbm, src_hbm, dst_hbm, ew_hbm,
          srcv, dstv, dotv, qr0, kr0, qr1, kr1, sq0, sk0, sq1, sk1):
        i32 = jnp.int32
        tid = (lax.axis_index("c").astype(i32) * i32(16)
               + lax.axis_index("s").astype(i32))
        base = tid * i32(ET)
        pltpu.sync_copy(src_hbm.at[pl.ds(base, ET)], srcv)
        pltpu.sync_copy(dst_hbm.at[pl.ds(base, ET)], dstv)

        lane = lax.iota(jnp.int32, 16)

        def issue(b, qrb, krb, sqb, skb):
            off = b * i32(16)
            pltpu.async_copy(q_hbm.at[dstv[pl.ds(off, 16)]], qrb, sqb)
            pltpu.async_copy(k_hbm.at[srcv[pl.ds(off, 16)]], krb, skb)

        def wait(qrb, krb, sqb, skb):
            dummy = q_hbm.at[pl.ds(i32(0), 16)]
            pltpu.make_async_copy(dummy, qrb, sqb).wait()
            pltpu.make_async_copy(dummy, krb, skb).wait()

        def compute(b, qrb, krb):
            dots = jnp.zeros((16,), jnp.float32)
            for r in range(16):
                def cb(cc, acc):
                    cco = cc * i32(16)
                    return acc + (qrb[r, pl.ds(cco, 16)]
                                  * krb[r, pl.ds(cco, 16)])
                accv = lax.fori_loop(i32(0), i32(W // 16), cb,
                                     jnp.zeros((16,), jnp.float32))
                dots = jnp.where(lane == r, jnp.sum(accv), dots)
            dotv[pl.ds(b * i32(16), 16)] = jnp.exp(dots * scale)

        issue(i32(0), qr0, kr0, sq0, sk0)
        issue(i32(1), qr1, kr1, sq1, sk1)

        def pair(p2, carry):
            b = p2 * i32(2)
            wait(qr0, kr0, sq0, sk0)
            compute(b, qr0, kr0)

            @pl.when(b + i32(2) < i32(NB))
            def _():
                issue(b + i32(2), qr0, kr0, sq0, sk0)
            wait(qr1, kr1, sq1, sk1)
            compute(b + i32(1), qr1, kr1)

            @pl.when(b + i32(3) < i32(NB))
            def _():
                issue(b + i32(3), qr1, kr1, sq1, sk1)
            return carry

        lax.fori_loop(i32(0), i32(NB // 2), pair, i32(0))
        if NB % 2:
            wait(qr0, kr0, sq0, sk0)
            compute(i32(NB - 1), qr0, kr0)
        pltpu.sync_copy(dotv, ew_hbm.at[pl.ds(base, ET)])

    return k


def _sc_scatter_common(C, gat):
    SLABS = max(C // 128, 1)
    SLABW = C // SLABS
    ET = EPAD // 16
    NB = ET // 16
    ZPT = SLABS * RS // 16

    scratch = [
        pltpu.VMEM((ET,), jnp.int32),
        pltpu.VMEM((ET,), jnp.int32),
        pltpu.VMEM((ET,), jnp.float32),
        pltpu.VMEM((NPAD,), jnp.float32),
        pltpu.VMEM((NPAD,), jnp.float32),
        pltpu.VMEM((ET + 16,), jnp.int32),
        pltpu.VMEM((ET + 16,), jnp.int32),
        pltpu.VMEM((ET + 16,), jnp.float32),
        pltpu.VMEM((SLABS * 16,), jnp.int32),
        pltpu.VMEM((R,), jnp.float32),
        pltpu.VMEM((16, C), jnp.float32),
        pltpu.VMEM((SLABS * 16, SLABW), jnp.float32),
        pltpu.VMEM((8, SLABW), jnp.float32),
        pltpu.VMEM((16 * STRIPE,), jnp.float32),
        pltpu.VMEM((STRIPE,), jnp.float32),
        pltpu.VMEM_SHARED((SLABS * RS, SLABW), jnp.float32),
        pltpu.VMEM_SHARED((16 * R,), jnp.float32),
        pltpu.SemaphoreType.DMA,
    ]

    def body(v_hbm, ew_hbm, asrc_hbm, adst_hbm, src_hbm, dst_hbm,
             numer_hbm, denom_hbm,
             srcv, dstv, ewv, asv, adv, ssel, ldsel, wsel, lidx, bins,
             rows, rows2, zbuf, stg, dnv, acc_sh, stage_sh, sem):
        i32 = jnp.int32
        cid = lax.axis_index("c").astype(i32)
        sid = lax.axis_index("s").astype(i32)
        base = sid * i32(ET)
        pltpu.sync_copy(src_hbm.at[pl.ds(base, ET)], srcv)
        pltpu.sync_copy(dst_hbm.at[pl.ds(base, ET)], dstv)
        if gat:
            pltpu.sync_copy(asrc_hbm, asv)
            pltpu.sync_copy(adst_hbm, adv)
        else:
            pltpu.sync_copy(ew_hbm.at[pl.ds(base, ET)], ewv)
        z16 = jnp.zeros((16,), jnp.float32)

        def zzr(r, carry):
            def zzc(cc, c2):
                zbuf[r, pl.ds(cc * i32(16), 16)] = z16
                return c2
            return lax.fori_loop(i32(0), i32(SLABW // 16), zzc, carry)
        lax.fori_loop(i32(0), i32(8), zzr, i32(0))

        def selfw(g0):
            a = asv[pl.ds(g0, 16)] + adv[pl.ds(g0, 16)]
            a = jnp.where(a > 0, a, 0.2 * a)
            return jnp.exp(a)

        def scale16(w, r):
            for si in range(SLABS):
                def sc_(cc, c3):
                    cco = cc * i32(16)
                    rows2[si * 16 + r, pl.ds(cco, 16)] = (
                        rows[r, pl.ds(i32(si * SLABW) + cco, 16)] * w)
                    return c3
                lax.fori_loop(i32(0), i32(SLABW // 16), sc_, i32(0))

        def chunk(kk, carry0):
            lo = (cid * i32(CPS) + kk) * i32(R)
            if not gat:
                def za(j, c2):
                    pltpu.sync_copy(
                        zbuf, acc_sh.at[pl.ds(sid * i32(ZPT) + j * i32(8), 8)])
                    return c2
                lax.fori_loop(i32(0), i32(ZPT // 8), za, i32(0))
            else:
                for j in range(STRIPE // 16):
                    g0 = lo + sid * i32(STRIPE) + i32(j * 16)
                    pltpu.sync_copy(v_hbm.at[pl.ds(g0, 16)], rows)
                    swv = selfw(g0)
                    for r in range(16):
                        scale16(swv[r], r)
                    for s in range(SLABS):
                        pltpu.sync_copy(
                            rows2.at[pl.ds(i32(s * 16), 16)],
                            acc_sh.at[pl.ds(i32(s * RS) + sid * i32(STRIPE)
                                            + i32(j * 16), 16)])

                @pl.when(sid == i32(0))
                def _():
                    for s in range(SLABS):
                        for j2 in range(2):
                            pltpu.sync_copy(
                                zbuf,
                                acc_sh.at[pl.ds(i32(s * RS + R + j2 * 8), 8)])

            def zb(i, c2):
                bins[pl.ds(i * i32(16), 16)] = z16
                return c2
            lax.fori_loop(i32(0), i32(R // 16), zb, i32(0))
            plsc.subcore_barrier()

            def scan(b, cnt):
                off = b * i32(16)
                s16 = srcv[pl.ds(off, 16)]
                d16 = dstv[pl.ds(off, 16)]
                if gat:
                    a = (plsc.load_gather(asv, [s16])
                         + plsc.load_gather(adv, [d16]))
                    a = jnp.where(a > 0, a, 0.2 * a)
                    w16 = jnp.exp(a)
                else:
                    w16 = ewv[pl.ds(off, 16)]
                ld = d16 - lo
                m = (d16 >= lo) & (d16 < lo + i32(R))
                plsc.addupdate_scatter(bins, [ld], w16, mask=m)
                plsc.store_compressed(ssel.at[pl.ds(cnt, 16)], s16, mask=m)
                plsc.store_compressed(ldsel.at[pl.ds(cnt, 16)], ld, mask=m)
                plsc.store_compressed(wsel.at[pl.ds(cnt, 16)], w16, mask=m)
                return cnt + jnp.sum(m.astype(jnp.int32), dtype=jnp.int32)
            cnt = lax.fori_loop(i32(0), i32(NB), scan, i32(0))

            ssel[pl.ds(cnt, 16)] = jnp.zeros((16,), jnp.int32)
            ldsel[pl.ds(cnt, 16)] = jnp.full((16,), R, jnp.int32)
            wsel[pl.ds(cnt, 16)] = z16

            def proc(j, c2):
                bs = j * i32(16)
                sidx = ssel[pl.ds(bs, 16)]
                ldv = ldsel[pl.ds(bs, 16)]
                wv = wsel[pl.ds(bs, 16)]
                pltpu.async_copy(v_hbm.at[sidx], rows, sem).wait()
                for r in range(16):
                    scale16(wv[r], r)
                for s in range(SLABS):
                    lidx[pl.ds(i32(s * 16), 16)] = ldv + i32(s * RS)
                pltpu.sync_copy(rows2, acc_sh.at[lidx], add=True)
                return c2
            nblk = (cnt + i32(15)) // i32(16)
            lax.fori_loop(i32(0), nblk, proc, i32(0))
            plsc.subcore_barrier()

            pltpu.sync_copy(bins, stage_sh.at[pl.ds(sid * i32(R), R)])
            plsc.subcore_barrier()
            for r in range(16):
                pltpu.sync_copy(
                    stage_sh.at[pl.ds(i32(r * R) + sid * i32(STRIPE), STRIPE)],
                    stg.at[pl.ds(i32(r * STRIPE), STRIPE)])

            def dred(bb, c2):
                acc = jnp.zeros((16,), jnp.float32)
                for r in range(16):
                    acc = acc + stg[pl.ds(i32(r * STRIPE) + bb * i32(16), 16)]
                if gat:
                    acc = acc + selfw(lo + sid * i32(STRIPE) + bb * i32(16))
                dnv[pl.ds(bb * i32(16), 16)] = acc
                return c2
            lax.fori_loop(i32(0), i32(STRIPE // 16), dred, i32(0))
            wo = lo + sid * i32(STRIPE)
            pltpu.sync_copy(dnv, denom_hbm.at[pl.ds(wo, STRIPE)])
            for s in range(SLABS):
                pltpu.sync_copy(
                    acc_sh.at[pl.ds(i32(s * RS) + sid * i32(STRIPE), STRIPE)],
                    numer_hbm.at[i32(s), pl.ds(wo, STRIPE)])
            plsc.subcore_barrier()
            return carry0

        lax.fori_loop(jnp.int32(0), jnp.int32(CPS), chunk, jnp.int32(0))

    out_type = (jax.ShapeDtypeStruct((SLABS, NPAD, SLABW), jnp.float32),
                jax.ShapeDtypeStruct((NPAD,), jnp.float32))
    return functools.partial(
        pl.kernel,
        out_type=out_type,
        mesh=plsc.VectorSubcoreMesh(**_MESH),
        compiler_params=pltpu.CompilerParams(needs_layout_passes=False),
        scratch_types=scratch,
    )(body)


def _sc_alpha_call(q, kmat, src, dst, C, W):
    return _make_sc_alpha(C, W)(q, kmat, src, dst)


def _sc_scatter_call(v, ew, src, dst, C):
    zeros_n = jnp.zeros((NPAD,), jnp.float32)
    return _sc_scatter_common(C, gat=False)(v, ew, zeros_n, zeros_n, src, dst)


def _sc_gat_call(h, asrc, adst, src, dst, C):
    zeros_e = jnp.zeros((EPAD,), jnp.float32)
    return _sc_scatter_common(C, gat=True)(h, zeros_e, asrc, adst, src, dst)


def kernel(x, edge_index, params):
    (q1W, q1b, k1W, k1b, v1W, v1b, s1W, s1b,
     q2W, q2b, k2W, k2b, v2W, v2b, s2W, s2b,
     g1W, g1as, g1ad, g1b,
     g2W, g2as, g2ad, g2b,
     f1W, f1b, f2W, f2b, f3W, f3b, f4W, f4b) = params

    f32 = jnp.float32
    src = edge_index[0].astype(jnp.int32)
    dst = edge_index[1].astype(jnp.int32)
    pad_e = EPAD - E
    src_p = jnp.concatenate([src, jnp.zeros((pad_e,), jnp.int32)])
    dst_p = jnp.concatenate([dst, jnp.full((pad_e,), SENT, jnp.int32)])

    xp = jnp.pad(x.astype(f32), ((0, NPAD - N), (0, 0)))

    was1 = g1as.reshape(-1) @ g1W
    wad1 = g1ad.reshape(-1) @ g1W
    w1 = jnp.concatenate([q1W, k1W, v1W, s1W, g1W,
                          was1.reshape(1, -1), wad1.reshape(1, -1),
                          jnp.zeros((254, DIN), f32)], axis=0).T
    b1 = jnp.concatenate([q1b, k1b, v1b, s1b,
                          jnp.zeros((C1 + 256,), f32)])
    p1 = _mm(xp, w1, b1, bm=512, bn=384)
    q1 = p1[:, 0:C1]
    k1 = p1[:, C1:2 * C1]
    v1 = p1[:, 2 * C1:3 * C1]
    s1 = p1[:, 3 * C1:4 * C1]
    h1g = p1[:, 4 * C1:5 * C1]
    as1 = p1[:, 5 * C1]
    ad1 = p1[:, 5 * C1 + 1]

    ew1 = _sc_alpha_call(q1, k1, src_p, dst_p, C1, C1)
    numt1, dent1 = _sc_scatter_call(v1, ew1, src_p, dst_p, C1)
    numg1, deng1 = _sc_gat_call(h1g, as1, ad1, src_p, dst_p, C1)

    was2 = g2as.reshape(-1) @ g2W
    wad2 = g2ad.reshape(-1) @ g2W
    z64 = jnp.zeros((64, C1), f32)
    wa2 = jnp.concatenate([q2W, z64, k2W, z64, v2W, z64,
                           jnp.zeros((128, C1), f32), s2W, z64],
                          axis=0).T
    wb2 = jnp.concatenate([jnp.zeros((384, C1), f32), g2W,
                           jnp.zeros((128, C1), f32),
                           was2.reshape(1, -1), wad2.reshape(1, -1),
                           jnp.zeros((62, C1), f32)], axis=0).T
    zb = jnp.zeros((64,), f32)
    b2 = jnp.concatenate([q2b, zb, k2b, zb, v2b, zb,
                          jnp.zeros((128,), f32), s2b, zb])
    p2 = _fuse_mm(numt1, dent1.reshape(-1, 1), s1,
                  numg1, deng1.reshape(-1, 1),
                  wa2.reshape(8, 128, 640), wb2.reshape(8, 128, 640),
                  b2, bm=512)
    q2 = p2[:, 0:128]
    k2 = p2[:, 128:256]
    v2 = p2[:, 256:384]
    h2g = p2[:, 384:512]
    s2 = p2[:, 512:576]
    as2 = p2[:, 576]
    ad2 = p2[:, 577]

    ew2 = _sc_alpha_call(q2, k2, src_p, dst_p, C2, 128)
    numt2, dent2 = _sc_scatter_call(v2, ew2, src_p, dst_p, 128)
    numg2, deng2 = _sc_gat_call(h2g, as2, ad2, src_p, dst_p, 128)

    y = _fuse_final(numt2, dent2.reshape(-1, 1), s2,
                    numg2, deng2.reshape(-1, 1), bm=1024)

    g = y[:N].reshape(N // 420, 420 * C2)
    w4p = jnp.pad(f4W.T, ((0, 0), (0, 118)))
    b4p = jnp.pad(f4b, (0, 118))
    out = _mlp(g, f1W.T, f1b, f2W.T, f2b, f3W.T, f3b, w4p, b4p)
    return out[:, :10].astype(jnp.float64)


# speedup vs baseline: 1.1450x; 1.1450x over previous
import functools

import numpy as np
import jax
import jax.numpy as jnp
from jax import lax
from jax.experimental import pallas as pl
from jax.experimental.pallas import tpu as pltpu
from jax.experimental.pallas import tpu_sc as plsc

N = 10080
E = 40320
DIN = 2048
C1 = 1024
C2 = 64

NPAD = 10240
EPAD = 40448
SENT = 10200
R = 512
RS = R + 128
CPS = 10
STRIPE = R // 16

_MESH = dict(core_axis_name="c", subcore_axis_name="s")


def _z0():
    return jnp.int32(0)


def _mm_kernel(a_ref, w_ref, b_ref, o_ref):
    o_ref[...] = (jnp.dot(a_ref[...], w_ref[...],
                          preferred_element_type=jnp.float32) + b_ref[...])


def _mm(a, w, b, bm, bn):
    m, k = a.shape
    _, n = w.shape
    return pl.pallas_call(
        _mm_kernel,
        grid=(m // bm, n // bn),
        in_specs=[pl.BlockSpec((bm, k), lambda i, j: (i, _z0())),
                  pl.BlockSpec((k, bn), lambda i, j: (_z0(), j)),
                  pl.BlockSpec((1, bn), lambda i, j: (_z0(), j))],
        out_specs=pl.BlockSpec((bm, bn), lambda i, j: (i, j)),
        out_shape=jax.ShapeDtypeStruct((m, n), jnp.float32),
    )(a, w, b.reshape(1, n))


def _fuse_mm_kernel(nt_ref, dt_ref, s1_ref, ng_ref, dg_ref,
                    wa_ref, wb_ref, b_ref, o_ref):
    ns = nt_ref.shape[0]
    sw = nt_ref.shape[2]
    dt = jnp.maximum(dt_ref[...], 1e-30)
    dg = jnp.maximum(dg_ref[...], 1e-30)
    acc = jnp.zeros(o_ref.shape, jnp.float32)
    for s in range(ns):
        h1s = jnp.maximum(nt_ref[s] / dt + s1_ref[:, s * sw:(s + 1) * sw], 0.0)
        z1s = jnp.maximum(ng_ref[s] / dg, 0.0)
        acc = acc + jnp.dot(h1s, wa_ref[s], preferred_element_type=jnp.float32)
        acc = acc + jnp.dot(z1s, wb_ref[s], preferred_element_type=jnp.float32)
    o_ref[...] = acc + b_ref[...]


def _fuse_mm(nt, dt, s1, ng, dg, wa, wb, b, bm):
    ns, m, sw = nt.shape
    n = wa.shape[2]
    return pl.pallas_call(
        _fuse_mm_kernel,
        grid=(m // bm,),
        in_specs=[pl.BlockSpec((ns, bm, sw), lambda i: (_z0(), i, _z0())),
                  pl.BlockSpec((bm, 1), lambda i: (i, _z0())),
                  pl.BlockSpec((bm, ns * sw), lambda i: (i, _z0())),
                  pl.BlockSpec((ns, bm, sw), lambda i: (_z0(), i, _z0())),
                  pl.BlockSpec((bm, 1), lambda i: (i, _z0())),
                  pl.BlockSpec((ns, sw, n), lambda i: (_z0(), _z0(), _z0())),
                  pl.BlockSpec((ns, sw, n), lambda i: (_z0(), _z0(), _z0())),
                  pl.BlockSpec((1, n), lambda i: (_z0(), _z0()))],
        out_specs=pl.BlockSpec((bm, n), lambda i: (i, _z0())),
        out_shape=jax.ShapeDtypeStruct((m, n), jnp.float32),
    )(nt, dt, s1, ng, dg, wa, wb, b.reshape(1, n))


def _fuse_final_kernel(nt_ref, dt_ref, s2_ref, ng_ref, dg_ref, o_ref):
    kw = s2_ref.shape[1]
    h2 = jnp.maximum(nt_ref[0][:, :kw] / jnp.maximum(dt_ref[...], 1e-30)
                     + s2_ref[...], 0.0)
    z2 = jnp.maximum(ng_ref[0][:, :kw] / jnp.maximum(dg_ref[...], 1e-30), 0.0)
    o_ref[...] = h2 + z2


def _fuse_final(nt, dt, s2, ng, dg, bm):
    _, m, w = nt.shape
    k = s2.shape[1]
    return pl.pallas_call(
        _fuse_final_kernel,
        grid=(m // bm,),
        in_specs=[pl.BlockSpec((1, bm, w), lambda i: (_z0(), i, _z0())),
                  pl.BlockSpec((bm, 1), lambda i: (i, _z0())),
                  pl.BlockSpec((bm, k), lambda i: (i, _z0())),
                  pl.BlockSpec((1, bm, w), lambda i: (_z0(), i, _z0())),
                  pl.BlockSpec((bm, 1), lambda i: (i, _z0()))],
        out_specs=pl.BlockSpec((bm, k), lambda i: (i, _z0())),
        out_shape=jax.ShapeDtypeStruct((m, k), jnp.float32),
    )(nt, dt, s2, ng, dg)


def _mlp_kernel(g_ref, w1_ref, b1_ref, w2_ref, b2_ref, w3_ref, b3_ref,
                w4_ref, b4_ref, o_ref):
    f32 = jnp.float32
    g = jnp.maximum(jnp.dot(g_ref[...], w1_ref[...],
                            preferred_element_type=f32) + b1_ref[...], 0.0)
    g = jnp.maximum(jnp.dot(g, w2_ref[...],
                            preferred_element_type=f32) + b2_ref[...], 0.0)
    g = jnp.maximum(jnp.dot(g, w3_ref[...],
                            preferred_element_type=f32) + b3_ref[...], 0.0)
    o_ref[...] = jnp.dot(g, w4_ref[...],
                         preferred_element_type=f32) + b4_ref[...]


def _mlp(g, w1, b1, w2, b2, w3, b3, w4, b4):
    return pl.pallas_call(
        _mlp_kernel,
        out_shape=jax.ShapeDtypeStruct((g.shape[0], w4.shape[1]), jnp.float32),
    )(g, w1, b1.reshape(1, -1), w2, b2.reshape(1, -1),
      w3, b3.reshape(1, -1), w4, b4.reshape(1, -1))


def _make_sc_alpha(C, W):
    ET = EPAD // 32
    NB = ET // 16
    scale = 1.0 / float(np.sqrt(C))

    @functools.partial(
        pl.kernel,
        out_type=jax.ShapeDtypeStruct((EPAD,), jnp.float32),
        mesh=plsc.VectorSubcoreMesh(**_MESH),
        compiler_params=pltpu.CompilerParams(needs_layout_passes=False),
        scratch_types=[
            pltpu.VMEM((ET,), jnp.int32),
            pltpu.VMEM((ET,), jnp.int32),
            pltpu.VMEM((ET,), jnp.float32),
            pltpu.VMEM((16, W), jnp.float32),
            pltpu.VMEM((16, W), jnp.float32),
            pltpu.VMEM((16, W), jnp.float32),
            pltpu.VMEM((16, W), jnp.float32),
            pltpu.SemaphoreType.DMA,
            pltpu.SemaphoreType.DMA,
            pltpu.SemaphoreType.DMA,
            pltpu.SemaphoreType.DMA,
        ],
    )
    def k(q_hbm, k_hbm, src_hbm, dst_hbm, ew_hbm,
          srcv, dstv, dotv, qr0, kr0, qr1, kr1, sq0, sk0, sq1, sk1):
        i32 = jnp.int32
        tid = (lax.axis_index("c").astype(i32) * i32(16)
               + lax.axis_index("s").astype(i32))
        base = tid * i32(ET)
        pltpu.sync_copy(src_hbm.at[pl.ds(base, ET)], srcv)
        pltpu.sync_copy(dst_hbm.at[pl.ds(base, ET)], dstv)

        lane = lax.iota(jnp.int32, 16)

        def issue(b, qrb, krb, sqb, skb):
            off = b * i32(16)
            pltpu.async_copy(q_hbm.at[dstv[pl.ds(off, 16)]], qrb, sqb)
            pltpu.async_copy(k_hbm.at[srcv[pl.ds(off, 16)]], krb, skb)

        def wait(qrb, krb, sqb, skb):
            dummy = q_hbm.at[pl.ds(i32(0), 16)]
            pltpu.make_async_copy(dummy, qrb, sqb).wait()
            pltpu.make_async_copy(dummy, krb, skb).wait()

        def compute(b, qrb, krb):
            dots = jnp.zeros((16,), jnp.float32)
            for r in range(16):
                def cb(cc, acc):
                    cco = cc * i32(16)
                    return acc + (qrb[r, pl.ds(cco, 16)]
                                  * krb[r, pl.ds(cco, 16)])
                accv = lax.fori_loop(i32(0), i32(W // 16), cb,
                                     jnp.zeros((16,), jnp.float32))
                dots = jnp.where(lane == r, jnp.sum(accv), dots)
            dotv[pl.ds(b * i32(16), 16)] = jnp.exp(dots * scale)

        issue(i32(0), qr0, kr0, sq0, sk0)
        issue(i32(1), qr1, kr1, sq1, sk1)

        def pair(p2, carry):
            b = p2 * i32(2)
            wait(qr0, kr0, sq0, sk0)
            compute(b, qr0, kr0)

            @pl.when(b + i32(2) < i32(NB))
            def _():
                issue(b + i32(2), qr0, kr0, sq0, sk0)
            wait(qr1, kr1, sq1, sk1)
            compute(b + i32(1), qr1, kr1)

            @pl.when(b + i32(3) < i32(NB))
            def _():
                issue(b + i32(3), qr1, kr1, sq1, sk1)
            return carry

        lax.fori_loop(i32(0), i32(NB // 2), pair, i32(0))
        if NB % 2:
            wait(qr0, kr0, sq0, sk0)
            compute(i32(NB - 1), qr0, kr0)
        pltpu.sync_copy(dotv, ew_hbm.at[pl.ds(base, ET)])

    return k


def _sc_scatter_common(C, gat):
    SLABS = max(C // 128, 1)
    SLABW = C // SLABS
    if SLABS == 1:
        R_, CPS_ = NPAD // 2, 1
    else:
        R_, CPS_ = R, CPS
    RS_ = R_ + 128
    STRIPE_ = R_ // 16
    ET = EPAD // 16
    NB = ET // 16
    ZPT = SLABS * RS_ // 16
    assert ZPT % 8 == 0 and STRIPE_ % 16 == 0

    scratch = [
        pltpu.VMEM((ET,), jnp.int32),
        pltpu.VMEM((ET,), jnp.int32),
        pltpu.VMEM((ET,), jnp.float32),
        pltpu.VMEM((NPAD,), jnp.float32),
        pltpu.VMEM((NPAD,), jnp.float32),
        pltpu.VMEM((ET + 16,), jnp.int32),
        pltpu.VMEM((ET + 16,), jnp.int32),
        pltpu.VMEM((ET + 16,), jnp.float32),
        pltpu.VMEM((SLABS * 16,), jnp.int32),
        pltpu.VMEM((R_,), jnp.float32),
        pltpu.VMEM((16, C), jnp.float32),
        pltpu.VMEM((SLABS * 16, SLABW), jnp.float32),
        pltpu.VMEM((8, SLABW), jnp.float32),
        pltpu.VMEM((16 * STRIPE_,), jnp.float32),
        pltpu.VMEM((STRIPE_,), jnp.float32),
        pltpu.VMEM_SHARED((SLABS * RS_, SLABW), jnp.float32),
        pltpu.VMEM_SHARED((16 * R_,), jnp.float32),
        pltpu.SemaphoreType.DMA,
        pltpu.SemaphoreType.DMA,
    ]

    def body(v_hbm, ew_hbm, asrc_hbm, adst_hbm, src_hbm, dst_hbm,
             numer_hbm, denom_hbm,
             srcv, dstv, ewv, asv, adv, ssel, ldsel, wsel, lidx, bins,
             rows, rows2, zbuf, stg, dnv, acc_sh, stage_sh, sem, ssem):
        i32 = jnp.int32
        cid = lax.axis_index("c").astype(i32)
        sid = lax.axis_index("s").astype(i32)
        base = sid * i32(ET)
        pltpu.sync_copy(src_hbm.at[pl.ds(base, ET)], srcv)
        pltpu.sync_copy(dst_hbm.at[pl.ds(base, ET)], dstv)
        if gat:
            pltpu.sync_copy(asrc_hbm, asv)
            pltpu.sync_copy(adst_hbm, adv)
        else:
            pltpu.sync_copy(ew_hbm.at[pl.ds(base, ET)], ewv)
        z16 = jnp.zeros((16,), jnp.float32)

        def zzr(r, carry):
            def zzc(cc, c2):
                zbuf[r, pl.ds(cc * i32(16), 16)] = z16
                return c2
            return lax.fori_loop(i32(0), i32(SLABW // 16), zzc, carry)
        lax.fori_loop(i32(0), i32(8), zzr, i32(0))

        def selfw(g0):
            a = asv[pl.ds(g0, 16)] + adv[pl.ds(g0, 16)]
            a = jnp.where(a > 0, a, 0.2 * a)
            return jnp.exp(a)

        def scale16(w, r):
            for si in range(SLABS):
                def sc_(cc, c3):
                    cco = cc * i32(16)
                    rows2[si * 16 + r, pl.ds(cco, 16)] = (
                        rows[r, pl.ds(i32(si * SLABW) + cco, 16)] * w)
                    return c3
                lax.fori_loop(i32(0), i32(SLABW // 16), sc_, i32(0))

        def chunk(kk, carry0):
            lo = (cid * i32(CPS_) + kk) * i32(R_)
            if not gat:
                def za(j, c2):
                    pltpu.sync_copy(
                        zbuf, acc_sh.at[pl.ds(sid * i32(ZPT) + j * i32(8), 8)])
                    return c2
                lax.fori_loop(i32(0), i32(ZPT // 8), za, i32(0))
            else:
                for j in range(STRIPE_ // 16):
                    g0 = lo + sid * i32(STRIPE_) + i32(j * 16)
                    pltpu.sync_copy(v_hbm.at[pl.ds(g0, 16)], rows)
                    swv = selfw(g0)
                    for r in range(16):
                        scale16(swv[r], r)
                    for s in range(SLABS):
                        pltpu.sync_copy(
                            rows2.at[pl.ds(i32(s * 16), 16)],
                            acc_sh.at[pl.ds(i32(s * RS_) + sid * i32(STRIPE_)
                                            + i32(j * 16), 16)])

                @pl.when(sid == i32(0))
                def _():
                    for s in range(SLABS):
                        for j2 in range(2):
                            pltpu.sync_copy(
                                zbuf,
                                acc_sh.at[pl.ds(i32(s * RS_ + R_ + j2 * 8), 8)])

            def zb(i, c2):
                bins[pl.ds(i * i32(16), 16)] = z16
                return c2
            lax.fori_loop(i32(0), i32(R_ // 16), zb, i32(0))
            plsc.subcore_barrier()

            def scan(b, cnt):
                off = b * i32(16)
                s16 = srcv[pl.ds(off, 16)]
                d16 = dstv[pl.ds(off, 16)]
                if gat:
                    a = (plsc.load_gather(asv, [s16])
                         + plsc.load_gather(adv, [d16]))
                    a = jnp.where(a > 0, a, 0.2 * a)
                    w16 = jnp.exp(a)
                else:
                    w16 = ewv[pl.ds(off, 16)]
                ld = d16 - lo
                m = (d16 >= lo) & (d16 < lo + i32(R_))
                plsc.addupdate_scatter(bins, [ld], w16, mask=m)
                plsc.store_compressed(ssel.at[pl.ds(cnt, 16)], s16, mask=m)
                plsc.store_compressed(ldsel.at[pl.ds(cnt, 16)], ld, mask=m)
                plsc.store_compressed(wsel.at[pl.ds(cnt, 16)], w16, mask=m)
                return cnt + jnp.sum(m.astype(jnp.int32), dtype=jnp.int32)
            cnt = lax.fori_loop(i32(0), i32(NB), scan, i32(0))

            ssel[pl.ds(cnt, 16)] = jnp.zeros((16,), jnp.int32)
            ldsel[pl.ds(cnt, 16)] = jnp.full((16,), R_, jnp.int32)
            wsel[pl.ds(cnt, 16)] = z16

            nblk = (cnt + i32(15)) // i32(16)

            @pl.when(nblk > i32(0))
            def _():
                pltpu.async_copy(v_hbm.at[ssel[pl.ds(i32(0), 16)]], rows, sem)

            def proc(j, c2):
                bs = j * i32(16)
                ldv = ldsel[pl.ds(bs, 16)]
                wv = wsel[pl.ds(bs, 16)]
                pltpu.make_async_copy(
                    v_hbm.at[pl.ds(i32(0), 16)], rows, sem).wait()

                @pl.when(j > i32(0))
                def _():
                    pltpu.make_async_copy(rows2, acc_sh.at[lidx], ssem).wait()
                for r in range(16):
                    scale16(wv[r], r)
                for s in range(SLABS):
                    lidx[pl.ds(i32(s * 16), 16)] = ldv + i32(s * RS_)
                pltpu.async_copy(rows2, acc_sh.at[lidx], ssem, add=True)

                @pl.when(j + i32(1) < nblk)
                def _():
                    pltpu.async_copy(
                        v_hbm.at[ssel[pl.ds(bs + i32(16), 16)]], rows, sem)
                return c2
            lax.fori_loop(i32(0), nblk, proc, i32(0))

            @pl.when(nblk > i32(0))
            def _():
                pltpu.make_async_copy(rows2, acc_sh.at[lidx], ssem).wait()
            plsc.subcore_barrier()

            pltpu.sync_copy(bins, stage_sh.at[pl.ds(sid * i32(R_), R_)])
            plsc.subcore_barrier()
            for r in range(16):
                pltpu.sync_copy(
                    stage_sh.at[pl.ds(i32(r * R_) + sid * i32(STRIPE_), STRIPE_)],
                    stg.at[pl.ds(i32(r * STRIPE_), STRIPE_)])

            def dred(bb, c2):
                acc = jnp.zeros((16,), jnp.float32)
                for r in range(16):
                    acc = acc + stg[pl.ds(i32(r * STRIPE_) + bb * i32(16), 16)]
                if gat:
                    acc = acc + selfw(lo + sid * i32(STRIPE_) + bb * i32(16))
                dnv[pl.ds(bb * i32(16), 16)] = acc
                return c2
            lax.fori_loop(i32(0), i32(STRIPE_ // 16), dred, i32(0))
            wo = lo + sid * i32(STRIPE_)
            pltpu.sync_copy(dnv, denom_hbm.at[pl.ds(wo, STRIPE_)])
            for s in range(SLABS):
                pltpu.sync_copy(
                    acc_sh.at[pl.ds(i32(s * RS_) + sid * i32(STRIPE_), STRIPE_)],
                    numer_hbm.at[i32(s), pl.ds(wo, STRIPE_)])
            plsc.subcore_barrier()
            return carry0

        lax.fori_loop(jnp.int32(0), jnp.int32(CPS_), chunk, jnp.int32(0))

    out_type = (jax.ShapeDtypeStruct((SLABS, NPAD, SLABW), jnp.float32),
                jax.ShapeDtypeStruct((NPAD,), jnp.float32))
    return functools.partial(
        pl.kernel,
        out_type=out_type,
        mesh=plsc.VectorSubcoreMesh(**_MESH),
        compiler_params=pltpu.CompilerParams(needs_layout_passes=False),
        scratch_types=scratch,
    )(body)


def _sc_alpha_call(q, kmat, src, dst, C, W):
    return _make_sc_alpha(C, W)(q, kmat, src, dst)


def _sc_scatter_call(v, ew, src, dst, C):
    zeros_n = jnp.zeros((NPAD,), jnp.float32)
    return _sc_scatter_common(C, gat=False)(v, ew, zeros_n, zeros_n, src, dst)


def _sc_gat_call(h, asrc, adst, src, dst, C):
    zeros_e = jnp.zeros((EPAD,), jnp.float32)
    return _sc_scatter_common(C, gat=True)(h, zeros_e, asrc, adst, src, dst)


def kernel(x, edge_index, params):
    (q1W, q1b, k1W, k1b, v1W, v1b, s1W, s1b,
     q2W, q2b, k2W, k2b, v2W, v2b, s2W, s2b,
     g1W, g1as, g1ad, g1b,
     g2W, g2as, g2ad, g2b,
     f1W, f1b, f2W, f2b, f3W, f3b, f4W, f4b) = params

    f32 = jnp.float32
    src = edge_index[0].astype(jnp.int32)
    dst = edge_index[1].astype(jnp.int32)
    pad_e = EPAD - E
    src_p = jnp.concatenate([src, jnp.zeros((pad_e,), jnp.int32)])
    dst_p = jnp.concatenate([dst, jnp.full((pad_e,), SENT, jnp.int32)])

    xp = jnp.pad(x.astype(f32), ((0, NPAD - N), (0, 0)))

    was1 = g1as.reshape(-1) @ g1W
    wad1 = g1ad.reshape(-1) @ g1W
    w1 = jnp.concatenate([q1W, k1W, v1W, s1W, g1W,
                          was1.reshape(1, -1), wad1.reshape(1, -1),
                          jnp.zeros((254, DIN), f32)], axis=0).T
    b1 = jnp.concatenate([q1b, k1b, v1b, s1b,
                          jnp.zeros((C1 + 256,), f32)])
    p1 = _mm(xp, w1, b1, bm=512, bn=384)
    q1 = p1[:, 0:C1]
    k1 = p1[:, C1:2 * C1]
    v1 = p1[:, 2 * C1:3 * C1]
    s1 = p1[:, 3 * C1:4 * C1]
    h1g = p1[:, 4 * C1:5 * C1]
    as1 = p1[:, 5 * C1]
    ad1 = p1[:, 5 * C1 + 1]

    ew1 = _sc_alpha_call(q1, k1, src_p, dst_p, C1, C1)
    numt1, dent1 = _sc_scatter_call(v1, ew1, src_p, dst_p, C1)
    numg1, deng1 = _sc_gat_call(h1g, as1, ad1, src_p, dst_p, C1)

    was2 = g2as.reshape(-1) @ g2W
    wad2 = g2ad.reshape(-1) @ g2W
    z64 = jnp.zeros((64, C1), f32)
    wa2 = jnp.concatenate([q2W, z64, k2W, z64, v2W, z64,
                           jnp.zeros((128, C1), f32), s2W, z64],
                          axis=0).T
    wb2 = jnp.concatenate([jnp.zeros((384, C1), f32), g2W,
                           jnp.zeros((128, C1), f32),
                           was2.reshape(1, -1), wad2.reshape(1, -1),
                           jnp.zeros((62, C1), f32)], axis=0).T
    zb = jnp.zeros((64,), f32)
    b2 = jnp.concatenate([q2b, zb, k2b, zb, v2b, zb,
                          jnp.zeros((128,), f32), s2b, zb])
    p2 = _fuse_mm(numt1, dent1.reshape(-1, 1), s1,
                  numg1, deng1.reshape(-1, 1),
                  wa2.reshape(8, 128, 640), wb2.reshape(8, 128, 640),
                  b2, bm=512)
    q2 = p2[:, 0:128]
    k2 = p2[:, 128:256]
    v2 = p2[:, 256:384]
    h2g = p2[:, 384:512]
    s2 = p2[:, 512:576]
    as2 = p2[:, 576]
    ad2 = p2[:, 577]

    ew2 = _sc_alpha_call(q2, k2, src_p, dst_p, C2, 128)
    numt2, dent2 = _sc_scatter_call(v2, ew2, src_p, dst_p, 128)
    numg2, deng2 = _sc_gat_call(h2g, as2, ad2, src_p, dst_p, 128)

    y = _fuse_final(numt2, dent2.reshape(-1, 1), s2,
                    numg2, deng2.reshape(-1, 1), bm=1024)

    g = y[:N].reshape(N // 420, 420 * C2)
    w4p = jnp.pad(f4W.T, ((0, 0), (0, 118)))
    b4p = jnp.pad(f4b, (0, 118))
    out = _mlp(g, f1W.T, f1b, f2W.T, f2b, f3W.T, f3b, w4p, b4p)
    return out[:, :10].astype(jnp.float64)


# speedup vs baseline: 1.2607x; 1.1011x over previous
import functools

import numpy as np
import jax
import jax.numpy as jnp
from jax import lax
from jax.experimental import pallas as pl
from jax.experimental.pallas import tpu as pltpu
from jax.experimental.pallas import tpu_sc as plsc

N = 10080
E = 40320
DIN = 2048
C1 = 1024
C2 = 64

NPAD = 10240
EPAD = 40448
SENT = 10200
R = 512
RS = R + 128
CPS = 10
STRIPE = R // 16

_MESH = dict(core_axis_name="c", subcore_axis_name="s")


def _z0():
    return jnp.int32(0)


def _mm_kernel(a_ref, w_ref, b_ref, o_ref):
    o_ref[...] = (jnp.dot(a_ref[...], w_ref[...],
                          preferred_element_type=jnp.float32) + b_ref[...])


def _mm(a, w, b, bm, bn):
    m, k = a.shape
    _, n = w.shape
    return pl.pallas_call(
        _mm_kernel,
        grid=(m // bm, n // bn),
        in_specs=[pl.BlockSpec((bm, k), lambda i, j: (i, _z0())),
                  pl.BlockSpec((k, bn), lambda i, j: (_z0(), j)),
                  pl.BlockSpec((1, bn), lambda i, j: (_z0(), j))],
        out_specs=pl.BlockSpec((bm, bn), lambda i, j: (i, j)),
        out_shape=jax.ShapeDtypeStruct((m, n), jnp.float32),
    )(a, w, b.reshape(1, n))


def _fuse_mm_kernel(nt_ref, dt_ref, s1_ref, ng_ref, dg_ref,
                    wa_ref, wb_ref, b_ref, o_ref):
    ns = nt_ref.shape[0]
    sw = nt_ref.shape[2]
    dt = jnp.maximum(dt_ref[...], 1e-30)
    dg = jnp.maximum(dg_ref[...], 1e-30)
    acc = jnp.zeros(o_ref.shape, jnp.float32)
    for s in range(ns):
        h1s = jnp.maximum(nt_ref[s] / dt + s1_ref[:, s * sw:(s + 1) * sw], 0.0)
        z1s = jnp.maximum(ng_ref[s] / dg, 0.0)
        acc = acc + jnp.dot(h1s, wa_ref[s], preferred_element_type=jnp.float32)
        acc = acc + jnp.dot(z1s, wb_ref[s], preferred_element_type=jnp.float32)
    o_ref[...] = acc + b_ref[...]


def _fuse_mm(nt, dt, s1, ng, dg, wa, wb, b, bm):
    ns, m, sw = nt.shape
    n = wa.shape[2]
    return pl.pallas_call(
        _fuse_mm_kernel,
        grid=(m // bm,),
        in_specs=[pl.BlockSpec((ns, bm, sw), lambda i: (_z0(), i, _z0())),
                  pl.BlockSpec((bm, 1), lambda i: (i, _z0())),
                  pl.BlockSpec((bm, ns * sw), lambda i: (i, _z0())),
                  pl.BlockSpec((ns, bm, sw), lambda i: (_z0(), i, _z0())),
                  pl.BlockSpec((bm, 1), lambda i: (i, _z0())),
                  pl.BlockSpec((ns, sw, n), lambda i: (_z0(), _z0(), _z0())),
                  pl.BlockSpec((ns, sw, n), lambda i: (_z0(), _z0(), _z0())),
                  pl.BlockSpec((1, n), lambda i: (_z0(), _z0()))],
        out_specs=pl.BlockSpec((bm, n), lambda i: (i, _z0())),
        out_shape=jax.ShapeDtypeStruct((m, n), jnp.float32),
    )(nt, dt, s1, ng, dg, wa, wb, b.reshape(1, n))


def _fuse_final_kernel(nt_ref, dt_ref, s2_ref, ng_ref, dg_ref, o_ref):
    kw = s2_ref.shape[1]
    h2 = jnp.maximum(nt_ref[0][:, :kw] / jnp.maximum(dt_ref[...], 1e-30)
                     + s2_ref[...], 0.0)
    z2 = jnp.maximum(ng_ref[0][:, :kw] / jnp.maximum(dg_ref[...], 1e-30), 0.0)
    o_ref[...] = h2 + z2


def _fuse_final(nt, dt, s2, ng, dg, bm):
    _, m, w = nt.shape
    k = s2.shape[1]
    return pl.pallas_call(
        _fuse_final_kernel,
        grid=(m // bm,),
        in_specs=[pl.BlockSpec((1, bm, w), lambda i: (_z0(), i, _z0())),
                  pl.BlockSpec((bm, 1), lambda i: (i, _z0())),
                  pl.BlockSpec((bm, k), lambda i: (i, _z0())),
                  pl.BlockSpec((1, bm, w), lambda i: (_z0(), i, _z0())),
                  pl.BlockSpec((bm, 1), lambda i: (i, _z0()))],
        out_specs=pl.BlockSpec((bm, k), lambda i: (i, _z0())),
        out_shape=jax.ShapeDtypeStruct((m, k), jnp.float32),
    )(nt, dt, s2, ng, dg)


def _mlp_kernel(g_ref, w1_ref, b1_ref, w2_ref, b2_ref, w3_ref, b3_ref,
                w4_ref, b4_ref, o_ref):
    f32 = jnp.float32
    g = jnp.maximum(jnp.dot(g_ref[...], w1_ref[...],
                            preferred_element_type=f32) + b1_ref[...], 0.0)
    g = jnp.maximum(jnp.dot(g, w2_ref[...],
                            preferred_element_type=f32) + b2_ref[...], 0.0)
    g = jnp.maximum(jnp.dot(g, w3_ref[...],
                            preferred_element_type=f32) + b3_ref[...], 0.0)
    o_ref[...] = jnp.dot(g, w4_ref[...],
                         preferred_element_type=f32) + b4_ref[...]


def _mlp(g, w1, b1, w2, b2, w3, b3, w4, b4):
    return pl.pallas_call(
        _mlp_kernel,
        out_shape=jax.ShapeDtypeStruct((g.shape[0], w4.shape[1]), jnp.float32),
    )(g, w1, b1.reshape(1, -1), w2, b2.reshape(1, -1),
      w3, b3.reshape(1, -1), w4, b4.reshape(1, -1))


def _make_sc_alpha(C, W):
    ET = EPAD // 32
    NB = ET // 16
    scale = 1.0 / float(np.sqrt(C))

    @functools.partial(
        pl.kernel,
        out_type=jax.ShapeDtypeStruct((EPAD,), jnp.float32),
        mesh=plsc.VectorSubcoreMesh(**_MESH),
        compiler_params=pltpu.CompilerParams(needs_layout_passes=False),
        scratch_types=[
            pltpu.VMEM((ET,), jnp.int32),
            pltpu.VMEM((ET,), jnp.int32),
            pltpu.VMEM((ET,), jnp.float32),
            pltpu.VMEM((16, W), jnp.float32),
            pltpu.VMEM((16, W), jnp.float32),
            pltpu.VMEM((16, W), jnp.float32),
            pltpu.VMEM((16, W), jnp.float32),
            pltpu.SemaphoreType.DMA,
            pltpu.SemaphoreType.DMA,
            pltpu.SemaphoreType.DMA,
            pltpu.SemaphoreType.DMA,
        ],
    )
    def k(q_hbm, k_hbm, src_hbm, dst_hbm, ew_hbm,
          srcv, dstv, dotv, qr0, kr0, qr1, kr1, sq0, sk0, sq1, sk1):
        i32 = jnp.int32
        tid = (lax.axis_index("c").astype(i32) * i32(16)
               + lax.axis_index("s").astype(i32))
        base = tid * i32(ET)
        pltpu.sync_copy(src_hbm.at[pl.ds(base, ET)], srcv)
        pltpu.sync_copy(dst_hbm.at[pl.ds(base, ET)], dstv)

        lane = lax.iota(jnp.int32, 16)

        def issue(b, qrb, krb, sqb, skb):
            off = b * i32(16)
            pltpu.async_copy(q_hbm.at[dstv[pl.ds(off, 16)]], qrb, sqb)
            pltpu.async_copy(k_hbm.at[srcv[pl.ds(off, 16)]], krb, skb)

        def wait(qrb, krb, sqb, skb):
            dummy = q_hbm.at[pl.ds(i32(0), 16)]
            pltpu.make_async_copy(dummy, qrb, sqb).wait()
            pltpu.make_async_copy(dummy, krb, skb).wait()

        def compute(b, qrb, krb):
            dots = jnp.zeros((16,), jnp.float32)
            for r in range(16):
                def cb(cc, acc):
                    cco = cc * i32(16)
                    return acc + (qrb[r, pl.ds(cco, 16)]
                                  * krb[r, pl.ds(cco, 16)])
                accv = lax.fori_loop(i32(0), i32(W // 16), cb,
                                     jnp.zeros((16,), jnp.float32))
                dots = jnp.where(lane == r, jnp.sum(accv), dots)
            dotv[pl.ds(b * i32(16), 16)] = jnp.exp(dots * scale)

        issue(i32(0), qr0, kr0, sq0, sk0)
        issue(i32(1), qr1, kr1, sq1, sk1)

        def pair(p2, carry):
            b = p2 * i32(2)
            wait(qr0, kr0, sq0, sk0)
            compute(b, qr0, kr0)

            @pl.when(b + i32(2) < i32(NB))
            def _():
                issue(b + i32(2), qr0, kr0, sq0, sk0)
            wait(qr1, kr1, sq1, sk1)
            compute(b + i32(1), qr1, kr1)

            @pl.when(b + i32(3) < i32(NB))
            def _():
                issue(b + i32(3), qr1, kr1, sq1, sk1)
            return carry

        lax.fori_loop(i32(0), i32(NB // 2), pair, i32(0))
        if NB % 2:
            wait(qr0, kr0, sq0, sk0)
            compute(i32(NB - 1), qr0, kr0)
        pltpu.sync_copy(dotv, ew_hbm.at[pl.ds(base, ET)])

    return k


def _sc_scatter_common(C, gat):
    SLABS = max(C // 128, 1)
    SLABW = C // SLABS
    if SLABS == 1:
        R_, CPS_ = NPAD // 2, 1
    else:
        R_, CPS_ = R, CPS
    RS_ = R_ + 128
    STRIPE_ = R_ // 16
    ET = EPAD // 16
    NB = ET // 16
    ZPT = SLABS * RS_ // 16
    assert ZPT % 8 == 0 and STRIPE_ % 16 == 0

    scratch = [
        pltpu.VMEM((ET,), jnp.int32),
        pltpu.VMEM((ET,), jnp.int32),
        pltpu.VMEM((ET,), jnp.float32),
        pltpu.VMEM((NPAD,), jnp.float32),
        pltpu.VMEM((NPAD,), jnp.float32),
        pltpu.VMEM((ET + 16,), jnp.int32),
        pltpu.VMEM((ET + 16,), jnp.int32),
        pltpu.VMEM((ET + 16,), jnp.float32),
        pltpu.VMEM((SLABS * 16,), jnp.int32),
        pltpu.VMEM((R_,), jnp.float32),
        pltpu.VMEM((16, C), jnp.float32),
        pltpu.VMEM((SLABS * 16, SLABW), jnp.float32),
        pltpu.VMEM((64, SLABW), jnp.float32),
        pltpu.VMEM((16 * STRIPE_,), jnp.float32),
        pltpu.VMEM((STRIPE_,), jnp.float32),
        pltpu.VMEM_SHARED((SLABS * RS_, SLABW), jnp.float32),
        pltpu.VMEM_SHARED((2 * 16 * R_,), jnp.float32),
        pltpu.SemaphoreType.DMA,
        pltpu.SemaphoreType.DMA,
    ]

    def body(v_hbm, ew_hbm, asrc_hbm, adst_hbm, src_hbm, dst_hbm,
             numer_hbm, denom_hbm,
             srcv, dstv, ewv, asv, adv, ssel, ldsel, wsel, lidx, bins,
             rows, rows2, zbuf, stg, dnv, acc_sh, stage_sh, sem, ssem):
        i32 = jnp.int32
        cid = lax.axis_index("c").astype(i32)
        sid = lax.axis_index("s").astype(i32)
        base = sid * i32(ET)
        pltpu.sync_copy(src_hbm.at[pl.ds(base, ET)], srcv)
        pltpu.sync_copy(dst_hbm.at[pl.ds(base, ET)], dstv)
        if gat:
            pltpu.sync_copy(asrc_hbm, asv)
            pltpu.sync_copy(adst_hbm, adv)
        else:
            pltpu.sync_copy(ew_hbm.at[pl.ds(base, ET)], ewv)
        z16 = jnp.zeros((16,), jnp.float32)

        def zzr(r, carry):
            def zzc(cc, c2):
                zbuf[r, pl.ds(cc * i32(16), 16)] = z16
                return c2
            return lax.fori_loop(i32(0), i32(SLABW // 16), zzc, carry)
        lax.fori_loop(i32(0), i32(64), zzr, i32(0))

        def selfw(g0):
            a = asv[pl.ds(g0, 16)] + adv[pl.ds(g0, 16)]
            a = jnp.where(a > 0, a, 0.2 * a)
            return jnp.exp(a)

        def scale16(w, r):
            for si in range(SLABS):
                def sc_(cc, c3):
                    cco = cc * i32(16)
                    rows2[si * 16 + r, pl.ds(cco, 16)] = (
                        rows[r, pl.ds(i32(si * SLABW) + cco, 16)] * w)
                    return c3
                lax.fori_loop(i32(0), i32(SLABW // 16), sc_, i32(0))

        def chunk(kk, carry0):
            lo = (cid * i32(CPS_) + kk) * i32(R_)
            if not gat:
                nf, rem = STRIPE_ // 64, STRIPE_ % 64
                for s in range(SLABS):
                    sb = i32(s * RS_) + sid * i32(STRIPE_)
                    for jf in range(nf):
                        pltpu.sync_copy(
                            zbuf, acc_sh.at[pl.ds(sb + i32(jf * 64), 64)])
                    if rem:
                        pltpu.sync_copy(
                            zbuf.at[pl.ds(i32(0), rem)],
                            acc_sh.at[pl.ds(sb + i32(nf * 64), rem)])
            else:
                for j in range(STRIPE_ // 16):
                    g0 = lo + sid * i32(STRIPE_) + i32(j * 16)
                    pltpu.sync_copy(v_hbm.at[pl.ds(g0, 16)], rows)
                    swv = selfw(g0)
                    for r in range(16):
                        scale16(swv[r], r)
                    for s in range(SLABS):
                        pltpu.sync_copy(
                            rows2.at[pl.ds(i32(s * 16), 16)],
                            acc_sh.at[pl.ds(i32(s * RS_) + sid * i32(STRIPE_)
                                            + i32(j * 16), 16)])

            def zb(i, c2):
                bins[pl.ds(i * i32(16), 16)] = z16
                return c2
            lax.fori_loop(i32(0), i32(R_ // 16), zb, i32(0))
            plsc.subcore_barrier()

            def scan(b, cnt):
                off = b * i32(16)
                s16 = srcv[pl.ds(off, 16)]
                d16 = dstv[pl.ds(off, 16)]
                if gat:
                    a = (plsc.load_gather(asv, [s16])
                         + plsc.load_gather(adv, [d16]))
                    a = jnp.where(a > 0, a, 0.2 * a)
                    w16 = jnp.exp(a)
                else:
                    w16 = ewv[pl.ds(off, 16)]
                ld = d16 - lo
                m = (d16 >= lo) & (d16 < lo + i32(R_))
                plsc.addupdate_scatter(bins, [ld], w16, mask=m)
                plsc.store_compressed(ssel.at[pl.ds(cnt, 16)], s16, mask=m)
                plsc.store_compressed(ldsel.at[pl.ds(cnt, 16)], ld, mask=m)
                plsc.store_compressed(wsel.at[pl.ds(cnt, 16)], w16, mask=m)
                return cnt + jnp.sum(m.astype(jnp.int32), dtype=jnp.int32)
            cnt = lax.fori_loop(i32(0), i32(NB), scan, i32(0))
            so = (kk % i32(2)) * i32(16 * R_)
            pltpu.sync_copy(bins, stage_sh.at[pl.ds(so + sid * i32(R_), R_)])

            ssel[pl.ds(cnt, 16)] = jnp.zeros((16,), jnp.int32)
            ldsel[pl.ds(cnt, 16)] = jnp.full((16,), R_, jnp.int32)
            wsel[pl.ds(cnt, 16)] = z16

            nblk = (cnt + i32(15)) // i32(16)

            @pl.when(nblk > i32(0))
            def _():
                pltpu.async_copy(v_hbm.at[ssel[pl.ds(i32(0), 16)]], rows, sem)

            def proc(j, c2):
                bs = j * i32(16)
                ldv = ldsel[pl.ds(bs, 16)]
                wv = wsel[pl.ds(bs, 16)]
                pltpu.make_async_copy(
                    v_hbm.at[pl.ds(i32(0), 16)], rows, sem).wait()

                @pl.when(j > i32(0))
                def _():
                    pltpu.make_async_copy(rows2, acc_sh.at[lidx], ssem).wait()
                for r in range(16):
                    scale16(wv[r], r)
                for s in range(SLABS):
                    lidx[pl.ds(i32(s * 16), 16)] = ldv + i32(s * RS_)
                pltpu.async_copy(rows2, acc_sh.at[lidx], ssem, add=True)

                @pl.when(j + i32(1) < nblk)
                def _():
                    pltpu.async_copy(
                        v_hbm.at[ssel[pl.ds(bs + i32(16), 16)]], rows, sem)
                return c2
            lax.fori_loop(i32(0), nblk, proc, i32(0))

            @pl.when(nblk > i32(0))
            def _():
                pltpu.make_async_copy(rows2, acc_sh.at[lidx], ssem).wait()
            plsc.subcore_barrier()
            for r in range(16):
                pltpu.sync_copy(
                    stage_sh.at[pl.ds(so + i32(r * R_) + sid * i32(STRIPE_),
                                      STRIPE_)],
                    stg.at[pl.ds(i32(r * STRIPE_), STRIPE_)])

            def dred(bb, c2):
                acc = jnp.zeros((16,), jnp.float32)
                for r in range(16):
                    acc = acc + stg[pl.ds(i32(r * STRIPE_) + bb * i32(16), 16)]
                if gat:
                    acc = acc + selfw(lo + sid * i32(STRIPE_) + bb * i32(16))
                dnv[pl.ds(bb * i32(16), 16)] = acc
                return c2
            lax.fori_loop(i32(0), i32(STRIPE_ // 16), dred, i32(0))
            wo = lo + sid * i32(STRIPE_)
            pltpu.sync_copy(dnv, denom_hbm.at[pl.ds(wo, STRIPE_)])
            for s in range(SLABS):
                pltpu.sync_copy(
                    acc_sh.at[pl.ds(i32(s * RS_) + sid * i32(STRIPE_), STRIPE_)],
                    numer_hbm.at[i32(s), pl.ds(wo, STRIPE_)])
            return carry0

        lax.fori_loop(jnp.int32(0), jnp.int32(CPS_), chunk, jnp.int32(0))

    out_type = (jax.ShapeDtypeStruct((SLABS, NPAD, SLABW), jnp.float32),
                jax.ShapeDtypeStruct((NPAD,), jnp.float32))
    return functools.partial(
        pl.kernel,
        out_type=out_type,
        mesh=plsc.VectorSubcoreMesh(**_MESH),
        compiler_params=pltpu.CompilerParams(needs_layout_passes=False),
        scratch_types=scratch,
    )(body)


def _sc_alpha_call(q, kmat, src, dst, C, W):
    return _make_sc_alpha(C, W)(q, kmat, src, dst)


def _sc_scatter_call(v, ew, src, dst, C):
    zeros_n = jnp.zeros((NPAD,), jnp.float32)
    return _sc_scatter_common(C, gat=False)(v, ew, zeros_n, zeros_n, src, dst)


def _sc_gat_call(h, asrc, adst, src, dst, C):
    zeros_e = jnp.zeros((EPAD,), jnp.float32)
    return _sc_scatter_common(C, gat=True)(h, zeros_e, asrc, adst, src, dst)


def kernel(x, edge_index, params):
    (q1W, q1b, k1W, k1b, v1W, v1b, s1W, s1b,
     q2W, q2b, k2W, k2b, v2W, v2b, s2W, s2b,
     g1W, g1as, g1ad, g1b,
     g2W, g2as, g2ad, g2b,
     f1W, f1b, f2W, f2b, f3W, f3b, f4W, f4b) = params

    f32 = jnp.float32
    src = edge_index[0].astype(jnp.int32)
    dst = edge_index[1].astype(jnp.int32)
    pad_e = EPAD - E
    src_p = jnp.concatenate([src, jnp.zeros((pad_e,), jnp.int32)])
    dst_p = jnp.concatenate([dst, jnp.full((pad_e,), SENT, jnp.int32)])

    xp = jnp.pad(x.astype(f32), ((0, NPAD - N), (0, 0)))

    was1 = g1as.reshape(-1) @ g1W
    wad1 = g1ad.reshape(-1) @ g1W
    w1qk = jnp.concatenate([q1W, k1W], axis=0).T
    b1qk = jnp.concatenate([q1b, k1b])
    p1qk = _mm(xp, w1qk, b1qk, bm=512, bn=512)
    q1 = p1qk[:, 0:C1]
    k1 = p1qk[:, C1:2 * C1]
    ew1 = _sc_alpha_call(q1, k1, src_p, dst_p, C1, C1)
    w1r = jnp.concatenate([v1W, s1W, g1W,
                           was1.reshape(1, -1), wad1.reshape(1, -1),
                           jnp.zeros((126, DIN), f32)], axis=0).T
    b1r = jnp.concatenate([v1b, s1b, jnp.zeros((C1 + 128,), f32)])
    p1 = _mm(xp, w1r, b1r, bm=512, bn=640)
    v1 = p1[:, 0:C1]
    s1 = p1[:, C1:2 * C1]
    h1g = p1[:, 2 * C1:3 * C1]
    as1 = p1[:, 3 * C1]
    ad1 = p1[:, 3 * C1 + 1]
    numt1, dent1 = _sc_scatter_call(v1, ew1, src_p, dst_p, C1)
    numg1, deng1 = _sc_gat_call(h1g, as1, ad1, src_p, dst_p, C1)

    was2 = g2as.reshape(-1) @ g2W
    wad2 = g2ad.reshape(-1) @ g2W
    z64 = jnp.zeros((64, C1), f32)
    wa2 = jnp.concatenate([q2W, z64, k2W, z64, v2W, z64,
                           jnp.zeros((128, C1), f32), s2W, z64],
                          axis=0).T
    wb2 = jnp.concatenate([jnp.zeros((384, C1), f32), g2W,
                           jnp.zeros((128, C1), f32),
                           was2.reshape(1, -1), wad2.reshape(1, -1),
                           jnp.zeros((62, C1), f32)], axis=0).T
    zb = jnp.zeros((64,), f32)
    b2 = jnp.concatenate([q2b, zb, k2b, zb, v2b, zb,
                          jnp.zeros((128,), f32), s2b, zb])
    p2 = _fuse_mm(numt1, dent1.reshape(-1, 1), s1,
                  numg1, deng1.reshape(-1, 1),
                  wa2.reshape(8, 128, 640), wb2.reshape(8, 128, 640),
                  b2, bm=512)
    q2 = p2[:, 0:128]
    k2 = p2[:, 128:256]
    v2 = p2[:, 256:384]
    h2g = p2[:, 384:512]
    s2 = p2[:, 512:576]
    as2 = p2[:, 576]
    ad2 = p2[:, 577]

    ew2 = _sc_alpha_call(q2, k2, src_p, dst_p, C2, 128)
    numt2, dent2 = _sc_scatter_call(v2, ew2, src_p, dst_p, 128)
    numg2, deng2 = _sc_gat_call(h2g, as2, ad2, src_p, dst_p, 128)

    y = _fuse_final(numt2, dent2.reshape(-1, 1), s2,
                    numg2, deng2.reshape(-1, 1), bm=1024)

    g = y[:N].reshape(N // 420, 420 * C2)
    w4p = jnp.pad(f4W.T, ((0, 0), (0, 118)))
    b4p = jnp.pad(f4b, (0, 118))
    out = _mlp(g, f1W.T, f1b, f2W.T, f2b, f3W.T, f3b, w4p, b4p)
    return out[:, :10].astype(jnp.float64)


# speedup vs baseline: 1.3249x; 1.0509x over previous
import functools

import numpy as np
import jax
import jax.numpy as jnp
from jax import lax
from jax.experimental import pallas as pl
from jax.experimental.pallas import tpu as pltpu
from jax.experimental.pallas import tpu_sc as plsc

N = 10080
E = 40320
DIN = 2048
C1 = 1024
C2 = 64

NPAD = 10240
EPAD = 40448
SENT = 10200
R = 512
RS = R + 128
CPS = 10
STRIPE = R // 16

_MESH = dict(core_axis_name="c", subcore_axis_name="s")


def _z0():
    return jnp.int32(0)


def _mm_kernel(a_ref, w_ref, b_ref, o_ref):
    o_ref[...] = (jnp.dot(a_ref[...], w_ref[...],
                          preferred_element_type=jnp.float32) + b_ref[...])


def _mm(a, w, b, bm, bn):
    m, k = a.shape
    _, n = w.shape
    return pl.pallas_call(
        _mm_kernel,
        grid=(m // bm, n // bn),
        in_specs=[pl.BlockSpec((bm, k), lambda i, j: (i, _z0())),
                  pl.BlockSpec((k, bn), lambda i, j: (_z0(), j)),
                  pl.BlockSpec((1, bn), lambda i, j: (_z0(), j))],
        out_specs=pl.BlockSpec((bm, bn), lambda i, j: (i, j)),
        out_shape=jax.ShapeDtypeStruct((m, n), jnp.float32),
    )(a, w, b.reshape(1, n))


def _fuse_mm_kernel(nt_ref, dt_ref, s1_ref, ng_ref, dg_ref,
                    wa_ref, wb_ref, b_ref, o_ref):
    ns = nt_ref.shape[0]
    sw = nt_ref.shape[2]
    dt = jnp.maximum(dt_ref[...], 1e-30)
    dg = jnp.maximum(dg_ref[...], 1e-30)
    acc = jnp.zeros(o_ref.shape, jnp.float32)
    for s in range(ns):
        h1s = jnp.maximum(nt_ref[s] / dt + s1_ref[:, s * sw:(s + 1) * sw], 0.0)
        z1s = jnp.maximum(ng_ref[s] / dg, 0.0)
        acc = acc + jnp.dot(h1s, wa_ref[s], preferred_element_type=jnp.float32)
        acc = acc + jnp.dot(z1s, wb_ref[s], preferred_element_type=jnp.float32)
    o_ref[...] = acc + b_ref[...]


def _fuse_mm(nt, dt, s1, ng, dg, wa, wb, b, bm):
    ns, m, sw = nt.shape
    n = wa.shape[2]
    return pl.pallas_call(
        _fuse_mm_kernel,
        grid=(m // bm,),
        in_specs=[pl.BlockSpec((ns, bm, sw), lambda i: (_z0(), i, _z0())),
                  pl.BlockSpec((bm, 1), lambda i: (i, _z0())),
                  pl.BlockSpec((bm, ns * sw), lambda i: (i, _z0())),
                  pl.BlockSpec((ns, bm, sw), lambda i: (_z0(), i, _z0())),
                  pl.BlockSpec((bm, 1), lambda i: (i, _z0())),
                  pl.BlockSpec((ns, sw, n), lambda i: (_z0(), _z0(), _z0())),
                  pl.BlockSpec((ns, sw, n), lambda i: (_z0(), _z0(), _z0())),
                  pl.BlockSpec((1, n), lambda i: (_z0(), _z0()))],
        out_specs=pl.BlockSpec((bm, n), lambda i: (i, _z0())),
        out_shape=jax.ShapeDtypeStruct((m, n), jnp.float32),
    )(nt, dt, s1, ng, dg, wa, wb, b.reshape(1, n))


def _fuse_final_kernel(nt_ref, dt_ref, s2_ref, ng_ref, dg_ref, o_ref):
    kw = s2_ref.shape[1]
    h2 = jnp.maximum(nt_ref[0][:, :kw] / jnp.maximum(dt_ref[...], 1e-30)
                     + s2_ref[...], 0.0)
    z2 = jnp.maximum(ng_ref[0][:, :kw] / jnp.maximum(dg_ref[...], 1e-30), 0.0)
    o_ref[...] = h2 + z2


def _fuse_final(nt, dt, s2, ng, dg, bm):
    _, m, w = nt.shape
    k = s2.shape[1]
    return pl.pallas_call(
        _fuse_final_kernel,
        grid=(m // bm,),
        in_specs=[pl.BlockSpec((1, bm, w), lambda i: (_z0(), i, _z0())),
                  pl.BlockSpec((bm, 1), lambda i: (i, _z0())),
                  pl.BlockSpec((bm, k), lambda i: (i, _z0())),
                  pl.BlockSpec((1, bm, w), lambda i: (_z0(), i, _z0())),
                  pl.BlockSpec((bm, 1), lambda i: (i, _z0()))],
        out_specs=pl.BlockSpec((bm, k), lambda i: (i, _z0())),
        out_shape=jax.ShapeDtypeStruct((m, k), jnp.float32),
    )(nt, dt, s2, ng, dg)


def _mlp_kernel(g_ref, w1_ref, b1_ref, w2_ref, b2_ref, w3_ref, b3_ref,
                w4_ref, b4_ref, o_ref):
    f32 = jnp.float32
    g = jnp.maximum(jnp.dot(g_ref[...], w1_ref[...],
                            preferred_element_type=f32) + b1_ref[...], 0.0)
    g = jnp.maximum(jnp.dot(g, w2_ref[...],
                            preferred_element_type=f32) + b2_ref[...], 0.0)
    g = jnp.maximum(jnp.dot(g, w3_ref[...],
                            preferred_element_type=f32) + b3_ref[...], 0.0)
    o_ref[...] = jnp.dot(g, w4_ref[...],
                         preferred_element_type=f32) + b4_ref[...]


def _mlp(g, w1, b1, w2, b2, w3, b3, w4, b4):
    return pl.pallas_call(
        _mlp_kernel,
        out_shape=jax.ShapeDtypeStruct((g.shape[0], w4.shape[1]), jnp.float32),
    )(g, w1, b1.reshape(1, -1), w2, b2.reshape(1, -1),
      w3, b3.reshape(1, -1), w4, b4.reshape(1, -1))


def _make_sc_alpha(C, W):
    ET = EPAD // 32
    NB = ET // 16
    scale = 1.0 / float(np.sqrt(C))

    @functools.partial(
        pl.kernel,
        out_type=jax.ShapeDtypeStruct((EPAD,), jnp.float32),
        mesh=plsc.VectorSubcoreMesh(**_MESH),
        compiler_params=pltpu.CompilerParams(needs_layout_passes=False),
        scratch_types=[
            pltpu.VMEM((ET,), jnp.int32),
            pltpu.VMEM((ET,), jnp.int32),
            pltpu.VMEM((ET,), jnp.float32),
            pltpu.VMEM((32, W), jnp.float32),
            pltpu.VMEM((32, W), jnp.float32),
            pltpu.VMEM((32,), jnp.int32),
            pltpu.VMEM((32,), jnp.int32),
            pltpu.SemaphoreType.DMA,
            pltpu.SemaphoreType.DMA,
        ],
    )
    def k(qk_hbm, src_hbm, dst_hbm, ew_hbm,
          srcv, dstv, dotv, qk0, qk1, idx0, idx1, sq0, sq1):
        i32 = jnp.int32
        tid = (lax.axis_index("c").astype(i32) * i32(16)
               + lax.axis_index("s").astype(i32))
        base = tid * i32(ET)
        pltpu.sync_copy(src_hbm.at[pl.ds(base, ET)], srcv)
        pltpu.sync_copy(dst_hbm.at[pl.ds(base, ET)], dstv)

        lane = lax.iota(jnp.int32, 16)

        def issue(b, qkb, idxb, sqb):
            off = b * i32(16)
            idxb[pl.ds(i32(0), 16)] = dstv[pl.ds(off, 16)] * i32(2)
            idxb[pl.ds(i32(16), 16)] = (srcv[pl.ds(off, 16)] * i32(2)
                                        + i32(1))
            pltpu.async_copy(qk_hbm.at[idxb], qkb, sqb)

        def wait(qkb, sqb):
            pltpu.make_async_copy(qk_hbm.at[pl.ds(i32(0), 32)], qkb,
                                  sqb).wait()

        def compute(b, qkb):
            dots = jnp.zeros((16,), jnp.float32)
            for r in range(16):
                def cb(cc, acc):
                    cco = cc * i32(16)
                    return acc + (qkb[r, pl.ds(cco, 16)]
                                  * qkb[16 + r, pl.ds(cco, 16)])
                accv = lax.fori_loop(i32(0), i32(W // 16), cb,
                                     jnp.zeros((16,), jnp.float32))
                dots = jnp.where(lane == r, jnp.sum(accv), dots)
            dotv[pl.ds(b * i32(16), 16)] = jnp.exp(dots * scale)

        issue(i32(0), qk0, idx0, sq0)
        issue(i32(1), qk1, idx1, sq1)

        def pair(p2, carry):
            b = p2 * i32(2)
            wait(qk0, sq0)
            compute(b, qk0)

            @pl.when(b + i32(2) < i32(NB))
            def _():
                issue(b + i32(2), qk0, idx0, sq0)
            wait(qk1, sq1)
            compute(b + i32(1), qk1)

            @pl.when(b + i32(3) < i32(NB))
            def _():
                issue(b + i32(3), qk1, idx1, sq1)
            return carry

        lax.fori_loop(i32(0), i32(NB // 2), pair, i32(0))
        if NB % 2:
            wait(qk0, sq0)
            compute(i32(NB - 1), qk0)
        pltpu.sync_copy(dotv, ew_hbm.at[pl.ds(base, ET)])

    return k


def _sc_scatter_common(C, gat):
    SLABS = max(C // 128, 1)
    SLABW = C // SLABS
    if SLABS == 1:
        R_, CPS_ = NPAD // 2, 1
    else:
        R_, CPS_ = R, CPS
    RS_ = R_ + 128
    STRIPE_ = R_ // 16
    ET = EPAD // 16
    NB = ET // 16
    ZPT = SLABS * RS_ // 16
    assert ZPT % 8 == 0 and STRIPE_ % 16 == 0

    scratch = [
        pltpu.VMEM((ET,), jnp.int32),
        pltpu.VMEM((ET,), jnp.int32),
        pltpu.VMEM((ET,), jnp.float32),
        pltpu.VMEM((NPAD,), jnp.float32),
        pltpu.VMEM((NPAD,), jnp.float32),
        pltpu.VMEM((ET + 512,), jnp.int32),
        pltpu.VMEM((ET + 512,), jnp.int32),
        pltpu.VMEM((ET + 512,), jnp.float32),
        pltpu.VMEM((SLABS * 16,), jnp.int32),
        pltpu.VMEM((R_,), jnp.float32),
        pltpu.VMEM((16, C), jnp.float32),
        pltpu.VMEM((SLABS * 16, SLABW), jnp.float32),
        pltpu.VMEM((64, SLABW), jnp.float32),
        pltpu.VMEM((16 * STRIPE_,), jnp.float32),
        pltpu.VMEM((STRIPE_,), jnp.float32),
        pltpu.VMEM_SHARED((SLABS * RS_, SLABW), jnp.float32),
        pltpu.VMEM_SHARED((2 * 16 * R_,), jnp.float32),
        pltpu.SemaphoreType.DMA,
        pltpu.SemaphoreType.DMA,
    ]

    def body(v_hbm, ew_hbm, asrc_hbm, adst_hbm, src_hbm, dst_hbm,
             numer_hbm, denom_hbm,
             srcv, dstv, ewv, asv, adv, ssel, ldsel, wsel, lidx, bins,
             rows, rows2, zbuf, stg, dnv, acc_sh, stage_sh, sem, ssem):
        i32 = jnp.int32
        cid = lax.axis_index("c").astype(i32)
        sid = lax.axis_index("s").astype(i32)
        base = sid * i32(ET)
        pltpu.sync_copy(src_hbm.at[pl.ds(base, ET)], srcv)
        pltpu.sync_copy(dst_hbm.at[pl.ds(base, ET)], dstv)
        if gat:
            pltpu.sync_copy(asrc_hbm, asv)
            pltpu.sync_copy(adst_hbm, adv)
        else:
            pltpu.sync_copy(ew_hbm.at[pl.ds(base, ET)], ewv)
        z16 = jnp.zeros((16,), jnp.float32)

        def zzr(r, carry):
            def zzc(cc, c2):
                zbuf[r, pl.ds(cc * i32(16), 16)] = z16
                return c2
            return lax.fori_loop(i32(0), i32(SLABW // 16), zzc, carry)
        lax.fori_loop(i32(0), i32(64), zzr, i32(0))

        def selfw(g0):
            a = asv[pl.ds(g0, 16)] + adv[pl.ds(g0, 16)]
            a = jnp.where(a > 0, a, 0.2 * a)
            return jnp.exp(a)

        def scale16(w, r):
            for si in range(SLABS):
                def sc_(cc, c3):
                    cco = cc * i32(16)
                    rows2[si * 16 + r, pl.ds(cco, 16)] = (
                        rows[r, pl.ds(i32(si * SLABW) + cco, 16)] * w)
                    return c3
                lax.fori_loop(i32(0), i32(SLABW // 16), sc_, i32(0))

        def chunk(kk, carry0):
            lo = (cid * i32(CPS_) + kk) * i32(R_)
            nf, rem = STRIPE_ // 64, STRIPE_ % 64
            for s in range(SLABS):
                sb = i32(s * RS_) + sid * i32(STRIPE_)
                for jf in range(nf):
                    pltpu.sync_copy(
                        zbuf, acc_sh.at[pl.ds(sb + i32(jf * 64), 64)])
                if rem:
                    pltpu.sync_copy(
                        zbuf.at[pl.ds(i32(0), rem)],
                        acc_sh.at[pl.ds(sb + i32(nf * 64), rem)])

            def zb(i, c2):
                bins[pl.ds(i * i32(16), 16)] = z16
                return c2
            lax.fori_loop(i32(0), i32(R_ // 16), zb, i32(0))
            plsc.subcore_barrier()

            def scan(b, cnt):
                off = b * i32(16)
                s16 = srcv[pl.ds(off, 16)]
                d16 = dstv[pl.ds(off, 16)]
                if gat:
                    a = (plsc.load_gather(asv, [s16])
                         + plsc.load_gather(adv, [d16]))
                    a = jnp.where(a > 0, a, 0.2 * a)
                    w16 = jnp.exp(a)
                else:
                    w16 = ewv[pl.ds(off, 16)]
                ld = d16 - lo
                m = (d16 >= lo) & (d16 < lo + i32(R_))
                plsc.addupdate_scatter(bins, [ld], w16, mask=m)
                plsc.store_compressed(ssel.at[pl.ds(cnt, 16)], s16, mask=m)
                plsc.store_compressed(ldsel.at[pl.ds(cnt, 16)], ld, mask=m)
                plsc.store_compressed(wsel.at[pl.ds(cnt, 16)], w16, mask=m)
                return cnt + jnp.sum(m.astype(jnp.int32), dtype=jnp.int32)
            cnt = lax.fori_loop(i32(0), i32(NB), scan, i32(0))
            if gat:
                lane = lax.iota(jnp.int32, 16)

                def selfapp(jj, cnt2):
                    g0 = lo + sid * i32(STRIPE_) + jj * i32(16)
                    rid = g0 + lane
                    ssel[pl.ds(cnt2, 16)] = rid
                    ldsel[pl.ds(cnt2, 16)] = rid - lo
                    wsel[pl.ds(cnt2, 16)] = selfw(g0)
                    return cnt2 + i32(16)
                cnt = lax.fori_loop(i32(0), i32(STRIPE_ // 16), selfapp, cnt)
            so = (kk % i32(2)) * i32(16 * R_)
            pltpu.sync_copy(bins, stage_sh.at[pl.ds(so + sid * i32(R_), R_)])

            ssel[pl.ds(cnt, 16)] = jnp.zeros((16,), jnp.int32)
            ldsel[pl.ds(cnt, 16)] = jnp.full((16,), R_, jnp.int32)
            wsel[pl.ds(cnt, 16)] = z16

            nblk = (cnt + i32(15)) // i32(16)

            @pl.when(nblk > i32(0))
            def _():
                pltpu.async_copy(v_hbm.at[ssel[pl.ds(i32(0), 16)]], rows, sem)

            def proc(j, c2):
                bs = j * i32(16)
                ldv = ldsel[pl.ds(bs, 16)]
                wv = wsel[pl.ds(bs, 16)]
                pltpu.make_async_copy(
                    v_hbm.at[pl.ds(i32(0), 16)], rows, sem).wait()

                @pl.when(j > i32(0))
                def _():
                    pltpu.make_async_copy(rows2, acc_sh.at[lidx], ssem).wait()
                for r in range(16):
                    scale16(wv[r], r)
                for s in range(SLABS):
                    lidx[pl.ds(i32(s * 16), 16)] = ldv + i32(s * RS_)
                pltpu.async_copy(rows2, acc_sh.at[lidx], ssem, add=True)

                @pl.when(j + i32(1) < nblk)
                def _():
                    pltpu.async_copy(
                        v_hbm.at[ssel[pl.ds(bs + i32(16), 16)]], rows, sem)
                return c2
            lax.fori_loop(i32(0), nblk, proc, i32(0))

            @pl.when(nblk > i32(0))
            def _():
                pltpu.make_async_copy(rows2, acc_sh.at[lidx], ssem).wait()
            plsc.subcore_barrier()
            for r in range(16):
                pltpu.sync_copy(
                    stage_sh.at[pl.ds(so + i32(r * R_) + sid * i32(STRIPE_),
                                      STRIPE_)],
                    stg.at[pl.ds(i32(r * STRIPE_), STRIPE_)])

            def dred(bb, c2):
                acc = jnp.zeros((16,), jnp.float32)
                for r in range(16):
                    acc = acc + stg[pl.ds(i32(r * STRIPE_) + bb * i32(16), 16)]
                if gat:
                    acc = acc + selfw(lo + sid * i32(STRIPE_) + bb * i32(16))
                dnv[pl.ds(bb * i32(16), 16)] = acc
                return c2
            lax.fori_loop(i32(0), i32(STRIPE_ // 16), dred, i32(0))
            wo = lo + sid * i32(STRIPE_)
            pltpu.sync_copy(dnv, denom_hbm.at[pl.ds(wo, STRIPE_)])
            for s in range(SLABS):
                pltpu.sync_copy(
                    acc_sh.at[pl.ds(i32(s * RS_) + sid * i32(STRIPE_), STRIPE_)],
                    numer_hbm.at[i32(s), pl.ds(wo, STRIPE_)])
            return carry0

        lax.fori_loop(jnp.int32(0), jnp.int32(CPS_), chunk, jnp.int32(0))

    out_type = (jax.ShapeDtypeStruct((SLABS, NPAD, SLABW), jnp.float32),
                jax.ShapeDtypeStruct((NPAD,), jnp.float32))
    return functools.partial(
        pl.kernel,
        out_type=out_type,
        mesh=plsc.VectorSubcoreMesh(**_MESH),
        compiler_params=pltpu.CompilerParams(needs_layout_passes=False),
        scratch_types=scratch,
    )(body)


def _sc_alpha_call(qk, src, dst, C, W):
    return _make_sc_alpha(C, W)(qk, src, dst)


def _sc_scatter_call(v, ew, src, dst, C):
    zeros_n = jnp.zeros((NPAD,), jnp.float32)
    return _sc_scatter_common(C, gat=False)(v, ew, zeros_n, zeros_n, src, dst)


def _sc_gat_call(h, asrc, adst, src, dst, C):
    zeros_e = jnp.zeros((EPAD,), jnp.float32)
    return _sc_scatter_common(C, gat=True)(h, zeros_e, asrc, adst, src, dst)


def kernel(x, edge_index, params):
    (q1W, q1b, k1W, k1b, v1W, v1b, s1W, s1b,
     q2W, q2b, k2W, k2b, v2W, v2b, s2W, s2b,
     g1W, g1as, g1ad, g1b,
     g2W, g2as, g2ad, g2b,
     f1W, f1b, f2W, f2b, f3W, f3b, f4W, f4b) = params

    f32 = jnp.float32
    src = edge_index[0].astype(jnp.int32)
    dst = edge_index[1].astype(jnp.int32)
    pad_e = EPAD - E
    src_p = jnp.concatenate([src, jnp.zeros((pad_e,), jnp.int32)])
    dst_p = jnp.concatenate([dst, jnp.full((pad_e,), SENT, jnp.int32)])

    xp = jnp.pad(x.astype(f32), ((0, NPAD - N), (0, 0)))

    was1 = g1as.reshape(-1) @ g1W
    wad1 = g1ad.reshape(-1) @ g1W
    w1qk = jnp.concatenate([q1W, k1W], axis=0).T
    b1qk = jnp.concatenate([q1b, k1b])
    p1qk = _mm(xp, w1qk, b1qk, bm=512, bn=512)
    ew1 = _sc_alpha_call(p1qk.reshape(2 * NPAD, C1), src_p, dst_p, C1, C1)
    w1r = jnp.concatenate([v1W, s1W, g1W,
                           was1.reshape(1, -1), wad1.reshape(1, -1),
                           jnp.zeros((126, DIN), f32)], axis=0).T
    b1r = jnp.concatenate([v1b, s1b, jnp.zeros((C1 + 128,), f32)])
    p1 = _mm(xp, w1r, b1r, bm=512, bn=640)
    v1 = p1[:, 0:C1]
    s1 = p1[:, C1:2 * C1]
    h1g = p1[:, 2 * C1:3 * C1]
    as1 = p1[:, 3 * C1]
    ad1 = p1[:, 3 * C1 + 1]
    numt1, dent1 = _sc_scatter_call(v1, ew1, src_p, dst_p, C1)
    numg1, deng1 = _sc_gat_call(h1g, as1, ad1, src_p, dst_p, C1)

    was2 = g2as.reshape(-1) @ g2W
    wad2 = g2ad.reshape(-1) @ g2W
    z64 = jnp.zeros((64, C1), f32)
    wa2 = jnp.concatenate([q2W, z64, k2W, z64, v2W, z64,
                           jnp.zeros((128, C1), f32), s2W, z64],
                          axis=0).T
    wb2 = jnp.concatenate([jnp.zeros((384, C1), f32), g2W,
                           jnp.zeros((128, C1), f32),
                           was2.reshape(1, -1), wad2.reshape(1, -1),
                           jnp.zeros((62, C1), f32)], axis=0).T
    zb = jnp.zeros((64,), f32)
    b2 = jnp.concatenate([q2b, zb, k2b, zb, v2b, zb,
                          jnp.zeros((128,), f32), s2b, zb])
    p2 = _fuse_mm(numt1, dent1.reshape(-1, 1), s1,
                  numg1, deng1.reshape(-1, 1),
                  wa2.reshape(8, 128, 640), wb2.reshape(8, 128, 640),
                  b2, bm=512)
    qk2 = p2[:, 0:256].reshape(2 * NPAD, 128)
    v2 = p2[:, 256:384]
    h2g = p2[:, 384:512]
    s2 = p2[:, 512:576]
    as2 = p2[:, 576]
    ad2 = p2[:, 577]

    ew2 = _sc_alpha_call(qk2, src_p, dst_p, C2, 128)
    numt2, dent2 = _sc_scatter_call(v2, ew2, src_p, dst_p, 128)
    numg2, deng2 = _sc_gat_call(h2g, as2, ad2, src_p, dst_p, 128)

    y = _fuse_final(numt2, dent2.reshape(-1, 1), s2,
                    numg2, deng2.reshape(-1, 1), bm=1024)

    g = y[:N].reshape(N // 420, 420 * C2)
    w4p = jnp.pad(f4W.T, ((0, 0), (0, 118)))
    b4p = jnp.pad(f4b, (0, 118))
    out = _mlp(g, f1W.T, f1b, f2W.T, f2b, f3W.T, f3b, w4p, b4p)
    return out[:, :10].astype(jnp.float64)


# speedup vs baseline: 1.4094x; 1.0638x over previous
import functools

import numpy as np
import jax
import jax.numpy as jnp
from jax import lax
from jax.experimental import pallas as pl
from jax.experimental.pallas import tpu as pltpu
from jax.experimental.pallas import tpu_sc as plsc

N = 10080
E = 40320
DIN = 2048
C1 = 1024
C2 = 64

NPAD = 10240
EPAD = 40448
SENT = 10200
R = 512
RS = R + 128
CPS = 10
STRIPE = R // 16

_MESH = dict(core_axis_name="c", subcore_axis_name="s")


def _z0():
    return jnp.int32(0)


def _mm_kernel(a_ref, w_ref, b_ref, o_ref):
    o_ref[...] = (jnp.dot(a_ref[...], w_ref[...],
                          preferred_element_type=jnp.float32) + b_ref[...])


def _mm(a, w, b, bm, bn):
    m, k = a.shape
    _, n = w.shape
    return pl.pallas_call(
        _mm_kernel,
        grid=(m // bm, n // bn),
        in_specs=[pl.BlockSpec((bm, k), lambda i, j: (i, _z0())),
                  pl.BlockSpec((k, bn), lambda i, j: (_z0(), j)),
                  pl.BlockSpec((1, bn), lambda i, j: (_z0(), j))],
        out_specs=pl.BlockSpec((bm, bn), lambda i, j: (i, j)),
        out_shape=jax.ShapeDtypeStruct((m, n), jnp.float32),
    )(a, w, b.reshape(1, n))


def _fuse_mm_kernel(nt_ref, dt_ref, s1_ref, ng_ref, dg_ref,
                    wa_ref, wb_ref, b_ref, o_ref):
    ns = nt_ref.shape[0]
    sw = nt_ref.shape[2]
    dt = jnp.maximum(dt_ref[...], 1e-30)
    dg = jnp.maximum(dg_ref[...], 1e-30)
    acc = jnp.zeros(o_ref.shape, jnp.float32)
    bf = jnp.bfloat16
    for s in range(ns):
        h1s = jnp.maximum(nt_ref[s] / dt + s1_ref[:, s * sw:(s + 1) * sw], 0.0)
        z1s = jnp.maximum(ng_ref[s] / dg, 0.0)
        acc = acc + jnp.dot(h1s.astype(bf), wa_ref[s].astype(bf),
                            preferred_element_type=jnp.float32)
        acc = acc + jnp.dot(z1s.astype(bf), wb_ref[s].astype(bf),
                            preferred_element_type=jnp.float32)
    o_ref[...] = acc + b_ref[...]


def _fuse_mm(nt, dt, s1, ng, dg, wa, wb, b, bm):
    ns, m, sw = nt.shape
    n = wa.shape[2]
    return pl.pallas_call(
        _fuse_mm_kernel,
        grid=(m // bm,),
        in_specs=[pl.BlockSpec((ns, bm, sw), lambda i: (_z0(), i, _z0())),
                  pl.BlockSpec((bm, 1), lambda i: (i, _z0())),
                  pl.BlockSpec((bm, ns * sw), lambda i: (i, _z0())),
                  pl.BlockSpec((ns, bm, sw), lambda i: (_z0(), i, _z0())),
                  pl.BlockSpec((bm, 1), lambda i: (i, _z0())),
                  pl.BlockSpec((ns, sw, n), lambda i: (_z0(), _z0(), _z0())),
                  pl.BlockSpec((ns, sw, n), lambda i: (_z0(), _z0(), _z0())),
                  pl.BlockSpec((1, n), lambda i: (_z0(), _z0()))],
        out_specs=pl.BlockSpec((bm, n), lambda i: (i, _z0())),
        out_shape=jax.ShapeDtypeStruct((m, n), jnp.float32),
    )(nt, dt, s1, ng, dg, wa, wb, b.reshape(1, n))


def _fuse_final_kernel(nt_ref, dt_ref, s2_ref, ng_ref, dg_ref, o_ref):
    kw = s2_ref.shape[1]
    h2 = jnp.maximum(nt_ref[0][:, :kw] / jnp.maximum(dt_ref[...], 1e-30)
                     + s2_ref[...], 0.0)
    z2 = jnp.maximum(ng_ref[0][:, :kw] / jnp.maximum(dg_ref[...], 1e-30), 0.0)
    o_ref[...] = h2 + z2


def _fuse_final(nt, dt, s2, ng, dg, bm):
    _, m, w = nt.shape
    k = s2.shape[1]
    return pl.pallas_call(
        _fuse_final_kernel,
        grid=(m // bm,),
        in_specs=[pl.BlockSpec((1, bm, w), lambda i: (_z0(), i, _z0())),
                  pl.BlockSpec((bm, 1), lambda i: (i, _z0())),
                  pl.BlockSpec((bm, k), lambda i: (i, _z0())),
                  pl.BlockSpec((1, bm, w), lambda i: (_z0(), i, _z0())),
                  pl.BlockSpec((bm, 1), lambda i: (i, _z0()))],
        out_specs=pl.BlockSpec((bm, k), lambda i: (i, _z0())),
        out_shape=jax.ShapeDtypeStruct((m, k), jnp.float32),
    )(nt, dt, s2, ng, dg)


def _mlp_kernel(g_ref, w1_ref, b1_ref, w2_ref, b2_ref, w3_ref, b3_ref,
                w4_ref, b4_ref, o_ref):
    f32 = jnp.float32
    g = jnp.maximum(jnp.dot(g_ref[...], w1_ref[...],
                            preferred_element_type=f32) + b1_ref[...], 0.0)
    g = jnp.maximum(jnp.dot(g, w2_ref[...],
                            preferred_element_type=f32) + b2_ref[...], 0.0)
    g = jnp.maximum(jnp.dot(g, w3_ref[...],
                            preferred_element_type=f32) + b3_ref[...], 0.0)
    o_ref[...] = jnp.dot(g, w4_ref[...],
                         preferred_element_type=f32) + b4_ref[...]


def _mlp(g, w1, b1, w2, b2, w3, b3, w4, b4):
    return pl.pallas_call(
        _mlp_kernel,
        out_shape=jax.ShapeDtypeStruct((g.shape[0], w4.shape[1]), jnp.float32),
    )(g, w1, b1.reshape(1, -1), w2, b2.reshape(1, -1),
      w3, b3.reshape(1, -1), w4, b4.reshape(1, -1))


def _make_sc_alpha(C, W):
    ET = EPAD // 32
    NB = ET // 16
    scale = 1.0 / float(np.sqrt(C))

    @functools.partial(
        pl.kernel,
        out_type=jax.ShapeDtypeStruct((EPAD,), jnp.float32),
        mesh=plsc.VectorSubcoreMesh(**_MESH),
        compiler_params=pltpu.CompilerParams(needs_layout_passes=False),
        scratch_types=[
            pltpu.VMEM((ET,), jnp.int32),
            pltpu.VMEM((ET,), jnp.int32),
            pltpu.VMEM((ET,), jnp.float32),
            pltpu.VMEM((32, W), jnp.float32),
            pltpu.VMEM((32, W), jnp.float32),
            pltpu.VMEM((32,), jnp.int32),
            pltpu.VMEM((32,), jnp.int32),
            pltpu.SemaphoreType.DMA,
            pltpu.SemaphoreType.DMA,
        ],
    )
    def k(qk_hbm, src_hbm, dst_hbm, ew_hbm,
          srcv, dstv, dotv, qk0, qk1, idx0, idx1, sq0, sq1):
        i32 = jnp.int32
        tid = (lax.axis_index("c").astype(i32) * i32(16)
               + lax.axis_index("s").astype(i32))
        base = tid * i32(ET)
        pltpu.sync_copy(src_hbm.at[pl.ds(base, ET)], srcv)
        pltpu.sync_copy(dst_hbm.at[pl.ds(base, ET)], dstv)

        lane = lax.iota(jnp.int32, 16)

        def issue(b, qkb, idxb, sqb):
            off = b * i32(16)
            idxb[pl.ds(i32(0), 16)] = dstv[pl.ds(off, 16)] * i32(2)
            idxb[pl.ds(i32(16), 16)] = (srcv[pl.ds(off, 16)] * i32(2)
                                        + i32(1))
            pltpu.async_copy(qk_hbm.at[idxb], qkb, sqb)

        def wait(qkb, sqb):
            pltpu.make_async_copy(qk_hbm.at[pl.ds(i32(0), 32)], qkb,
                                  sqb).wait()

        def compute(b, qkb):
            dots = jnp.zeros((16,), jnp.float32)
            for r in range(16):
                def cb(cc, acc):
                    cco = cc * i32(16)
                    return acc + (qkb[r, pl.ds(cco, 16)]
                                  * qkb[16 + r, pl.ds(cco, 16)])
                accv = lax.fori_loop(i32(0), i32(W // 16), cb,
                                     jnp.zeros((16,), jnp.float32))
                dots = jnp.where(lane == r, jnp.sum(accv), dots)
            dotv[pl.ds(b * i32(16), 16)] = jnp.exp(dots * scale)

        issue(i32(0), qk0, idx0, sq0)
        issue(i32(1), qk1, idx1, sq1)

        def pair(p2, carry):
            b = p2 * i32(2)
            wait(qk0, sq0)
            compute(b, qk0)

            @pl.when(b + i32(2) < i32(NB))
            def _():
                issue(b + i32(2), qk0, idx0, sq0)
            wait(qk1, sq1)
            compute(b + i32(1), qk1)

            @pl.when(b + i32(3) < i32(NB))
            def _():
                issue(b + i32(3), qk1, idx1, sq1)
            return carry

        lax.fori_loop(i32(0), i32(NB // 2), pair, i32(0))
        if NB % 2:
            wait(qk0, sq0)
            compute(i32(NB - 1), qk0)
        pltpu.sync_copy(dotv, ew_hbm.at[pl.ds(base, ET)])

    return k


def _sc_scatter_common(C, gat):
    SLABS = max(C // 128, 1)
    SLABW = C // SLABS
    if SLABS == 1:
        R_, CPS_ = NPAD // 2, 1
    else:
        R_, CPS_ = R, CPS
    RS_ = R_ + 128
    STRIPE_ = R_ // 16
    ET = EPAD // 16
    NB = ET // 16
    ZPT = SLABS * RS_ // 16
    assert ZPT % 8 == 0 and STRIPE_ % 16 == 0

    scratch = [
        pltpu.VMEM((ET,), jnp.int32),
        pltpu.VMEM((ET,), jnp.int32),
        pltpu.VMEM((ET,), jnp.float32),
        pltpu.VMEM((NPAD,), jnp.float32),
        pltpu.VMEM((NPAD,), jnp.float32),
        pltpu.VMEM((ET + 512,), jnp.int32),
        pltpu.VMEM((ET + 512,), jnp.int32),
        pltpu.VMEM((ET + 512,), jnp.float32),
        pltpu.VMEM((SLABS * 16,), jnp.int32),
        pltpu.VMEM((R_,), jnp.float32),
        pltpu.VMEM((16, C), jnp.float32),
        pltpu.VMEM((SLABS * 16, SLABW), jnp.float32),
        pltpu.VMEM((64, SLABW), jnp.float32),
        pltpu.VMEM((16 * STRIPE_,), jnp.float32),
        pltpu.VMEM((STRIPE_,), jnp.float32),
        pltpu.VMEM_SHARED((SLABS * RS_, SLABW), jnp.float32),
        pltpu.VMEM_SHARED((2 * 16 * R_,), jnp.float32),
        pltpu.SemaphoreType.DMA,
        pltpu.SemaphoreType.DMA,
    ]

    def body(v_hbm, ew_hbm, asrc_hbm, adst_hbm, src_hbm, dst_hbm,
             numer_hbm, denom_hbm,
             srcv, dstv, ewv, asv, adv, ssel, ldsel, wsel, lidx, bins,
             rows, rows2, zbuf, stg, dnv, acc_sh, stage_sh, sem, ssem):
        i32 = jnp.int32
        cid = lax.axis_index("c").astype(i32)
        sid = lax.axis_index("s").astype(i32)
        base = sid * i32(ET)
        pltpu.sync_copy(src_hbm.at[pl.ds(base, ET)], srcv)
        pltpu.sync_copy(dst_hbm.at[pl.ds(base, ET)], dstv)
        if gat:
            pltpu.sync_copy(asrc_hbm, asv)
            pltpu.sync_copy(adst_hbm, adv)
        else:
            pltpu.sync_copy(ew_hbm.at[pl.ds(base, ET)], ewv)
        z16 = jnp.zeros((16,), jnp.float32)

        def zzr(r, carry):
            def zzc(cc, c2):
                zbuf[r, pl.ds(cc * i32(16), 16)] = z16
                return c2
            return lax.fori_loop(i32(0), i32(SLABW // 16), zzc, carry)
        lax.fori_loop(i32(0), i32(64), zzr, i32(0))

        def selfw(g0):
            a = asv[pl.ds(g0, 16)] + adv[pl.ds(g0, 16)]
            a = jnp.where(a > 0, a, 0.2 * a)
            return jnp.exp(a)

        def scale16(w, r):
            for si in range(SLABS):
                def sc_(cc, c3):
                    cco = cc * i32(16)
                    rows2[si * 16 + r, pl.ds(cco, 16)] = (
                        rows[r, pl.ds(i32(si * SLABW) + cco, 16)] * w)
                    return c3
                lax.fori_loop(i32(0), i32(SLABW // 16), sc_, i32(0))

        def chunk(kk, carry0):
            lo = (cid * i32(CPS_) + kk) * i32(R_)
            nf, rem = STRIPE_ // 64, STRIPE_ % 64
            for s in range(SLABS):
                sb = i32(s * RS_) + sid * i32(STRIPE_)
                for jf in range(nf):
                    pltpu.sync_copy(
                        zbuf, acc_sh.at[pl.ds(sb + i32(jf * 64), 64)])
                if rem:
                    pltpu.sync_copy(
                        zbuf.at[pl.ds(i32(0), rem)],
                        acc_sh.at[pl.ds(sb + i32(nf * 64), rem)])

            def zb(i, c2):
                bins[pl.ds(i * i32(16), 16)] = z16
                return c2
            lax.fori_loop(i32(0), i32(R_ // 16), zb, i32(0))
            plsc.subcore_barrier()

            def scan(b, cnt):
                off = b * i32(16)
                s16 = srcv[pl.ds(off, 16)]
                d16 = dstv[pl.ds(off, 16)]
                if gat:
                    a = (plsc.load_gather(asv, [s16])
                         + plsc.load_gather(adv, [d16]))
                    a = jnp.where(a > 0, a, 0.2 * a)
                    w16 = jnp.exp(a)
                else:
                    w16 = ewv[pl.ds(off, 16)]
                ld = d16 - lo
                m = (d16 >= lo) & (d16 < lo + i32(R_))
                plsc.addupdate_scatter(bins, [ld], w16, mask=m)
                plsc.store_compressed(ssel.at[pl.ds(cnt, 16)], s16, mask=m)
                plsc.store_compressed(ldsel.at[pl.ds(cnt, 16)], ld, mask=m)
                plsc.store_compressed(wsel.at[pl.ds(cnt, 16)], w16, mask=m)
                return cnt + jnp.sum(m.astype(jnp.int32), dtype=jnp.int32)
            cnt = lax.fori_loop(i32(0), i32(NB), scan, i32(0))
            if gat:
                lane = lax.iota(jnp.int32, 16)

                def selfapp(jj, cnt2):
                    g0 = lo + sid * i32(STRIPE_) + jj * i32(16)
                    rid = g0 + lane
                    ssel[pl.ds(cnt2, 16)] = rid
                    ldsel[pl.ds(cnt2, 16)] = rid - lo
                    wsel[pl.ds(cnt2, 16)] = selfw(g0)
                    return cnt2 + i32(16)
                cnt = lax.fori_loop(i32(0), i32(STRIPE_ // 16), selfapp, cnt)
            so = (kk % i32(2)) * i32(16 * R_)
            pltpu.sync_copy(bins, stage_sh.at[pl.ds(so + sid * i32(R_), R_)])

            ssel[pl.ds(cnt, 16)] = jnp.zeros((16,), jnp.int32)
            ldsel[pl.ds(cnt, 16)] = jnp.full((16,), R_, jnp.int32)
            wsel[pl.ds(cnt, 16)] = z16

            nblk = (cnt + i32(15)) // i32(16)

            @pl.when(nblk > i32(0))
            def _():
                pltpu.async_copy(v_hbm.at[ssel[pl.ds(i32(0), 16)]], rows, sem)

            def proc(j, c2):
                bs = j * i32(16)
                ldv = ldsel[pl.ds(bs, 16)]
                wv = wsel[pl.ds(bs, 16)]
                pltpu.make_async_copy(
                    v_hbm.at[pl.ds(i32(0), 16)], rows, sem).wait()

                @pl.when(j > i32(0))
                def _():
                    pltpu.make_async_copy(rows2, acc_sh.at[lidx], ssem).wait()
                for r in range(16):
                    scale16(wv[r], r)
                for s in range(SLABS):
                    lidx[pl.ds(i32(s * 16), 16)] = ldv + i32(s * RS_)
                pltpu.async_copy(rows2, acc_sh.at[lidx], ssem, add=True)

                @pl.when(j + i32(1) < nblk)
                def _():
                    pltpu.async_copy(
                        v_hbm.at[ssel[pl.ds(bs + i32(16), 16)]], rows, sem)
                return c2
            lax.fori_loop(i32(0), nblk, proc, i32(0))

            @pl.when(nblk > i32(0))
            def _():
                pltpu.make_async_copy(rows2, acc_sh.at[lidx], ssem).wait()
            plsc.subcore_barrier()
            for r in range(16):
                pltpu.sync_copy(
                    stage_sh.at[pl.ds(so + i32(r * R_) + sid * i32(STRIPE_),
                                      STRIPE_)],
                    stg.at[pl.ds(i32(r * STRIPE_), STRIPE_)])

            def dred(bb, c2):
                acc = jnp.zeros((16,), jnp.float32)
                for r in range(16):
                    acc = acc + stg[pl.ds(i32(r * STRIPE_) + bb * i32(16), 16)]
                if gat:
                    acc = acc + selfw(lo + sid * i32(STRIPE_) + bb * i32(16))
                dnv[pl.ds(bb * i32(16), 16)] = acc
                return c2
            lax.fori_loop(i32(0), i32(STRIPE_ // 16), dred, i32(0))
            wo = lo + sid * i32(STRIPE_)
            pltpu.sync_copy(dnv, denom_hbm.at[pl.ds(wo, STRIPE_)])
            for s in range(SLABS):
                pltpu.sync_copy(
                    acc_sh.at[pl.ds(i32(s * RS_) + sid * i32(STRIPE_), STRIPE_)],
                    numer_hbm.at[i32(s), pl.ds(wo, STRIPE_)])
            return carry0

        lax.fori_loop(jnp.int32(0), jnp.int32(CPS_), chunk, jnp.int32(0))

    out_type = (jax.ShapeDtypeStruct((SLABS, NPAD, SLABW), jnp.float32),
                jax.ShapeDtypeStruct((NPAD,), jnp.float32))
    return functools.partial(
        pl.kernel,
        out_type=out_type,
        mesh=plsc.VectorSubcoreMesh(**_MESH),
        compiler_params=pltpu.CompilerParams(needs_layout_passes=False),
        scratch_types=scratch,
    )(body)


def _sc_alpha_call(qk, src, dst, C, W):
    return _make_sc_alpha(C, W)(qk, src, dst)


def _sc_scatter_call(v, ew, src, dst, C):
    zeros_n = jnp.zeros((NPAD,), jnp.float32)
    return _sc_scatter_common(C, gat=False)(v, ew, zeros_n, zeros_n, src, dst)


def _sc_gat_call(h, asrc, adst, src, dst, C):
    zeros_e = jnp.zeros((EPAD,), jnp.float32)
    return _sc_scatter_common(C, gat=True)(h, zeros_e, asrc, adst, src, dst)


def kernel(x, edge_index, params):
    (q1W, q1b, k1W, k1b, v1W, v1b, s1W, s1b,
     q2W, q2b, k2W, k2b, v2W, v2b, s2W, s2b,
     g1W, g1as, g1ad, g1b,
     g2W, g2as, g2ad, g2b,
     f1W, f1b, f2W, f2b, f3W, f3b, f4W, f4b) = params

    f32 = jnp.float32
    src = edge_index[0].astype(jnp.int32)
    dst = edge_index[1].astype(jnp.int32)
    pad_e = EPAD - E
    src_p = jnp.concatenate([src, jnp.zeros((pad_e,), jnp.int32)])
    dst_p = jnp.concatenate([dst, jnp.full((pad_e,), SENT, jnp.int32)])

    xp = jnp.pad(x.astype(f32), ((0, NPAD - N), (0, 0)))

    was1 = g1as.reshape(-1) @ g1W
    wad1 = g1ad.reshape(-1) @ g1W
    w1qk = jnp.concatenate([q1W, k1W], axis=0).T
    b1qk = jnp.concatenate([q1b, k1b])
    xb = xp.astype(jnp.bfloat16)
    p1qk = _mm(xb, w1qk.astype(jnp.bfloat16), b1qk, bm=512, bn=512)
    ew1 = _sc_alpha_call(p1qk.reshape(2 * NPAD, C1), src_p, dst_p, C1, C1)
    w1r = jnp.concatenate([v1W, s1W, g1W,
                           was1.reshape(1, -1), wad1.reshape(1, -1),
                           jnp.zeros((126, DIN), f32)], axis=0).T
    b1r = jnp.concatenate([v1b, s1b, jnp.zeros((C1 + 128,), f32)])
    p1 = _mm(xb, w1r.astype(jnp.bfloat16), b1r, bm=512, bn=640)
    v1 = p1[:, 0:C1]
    s1 = p1[:, C1:2 * C1]
    h1g = p1[:, 2 * C1:3 * C1]
    as1 = p1[:, 3 * C1]
    ad1 = p1[:, 3 * C1 + 1]
    numt1, dent1 = _sc_scatter_call(v1, ew1, src_p, dst_p, C1)
    numg1, deng1 = _sc_gat_call(h1g, as1, ad1, src_p, dst_p, C1)

    was2 = g2as.reshape(-1) @ g2W
    wad2 = g2ad.reshape(-1) @ g2W
    z64 = jnp.zeros((64, C1), f32)
    wa2 = jnp.concatenate([q2W, z64, k2W, z64, v2W, z64,
                           jnp.zeros((128, C1), f32), s2W, z64],
                          axis=0).T
    wb2 = jnp.concatenate([jnp.zeros((384, C1), f32), g2W,
                           jnp.zeros((128, C1), f32),
                           was2.reshape(1, -1), wad2.reshape(1, -1),
                           jnp.zeros((62, C1), f32)], axis=0).T
    zb = jnp.zeros((64,), f32)
    b2 = jnp.concatenate([q2b, zb, k2b, zb, v2b, zb,
                          jnp.zeros((128,), f32), s2b, zb])
    p2 = _fuse_mm(numt1, dent1.reshape(-1, 1), s1,
                  numg1, deng1.reshape(-1, 1),
                  wa2.reshape(8, 128, 640), wb2.reshape(8, 128, 640),
                  b2, bm=512)
    qk2 = p2[:, 0:256].reshape(2 * NPAD, 128)
    v2 = p2[:, 256:384]
    h2g = p2[:, 384:512]
    s2 = p2[:, 512:576]
    as2 = p2[:, 576]
    ad2 = p2[:, 577]

    ew2 = _sc_alpha_call(qk2, src_p, dst_p, C2, 128)
    numt2, dent2 = _sc_scatter_call(v2, ew2, src_p, dst_p, 128)
    numg2, deng2 = _sc_gat_call(h2g, as2, ad2, src_p, dst_p, 128)

    y = _fuse_final(numt2, dent2.reshape(-1, 1), s2,
                    numg2, deng2.reshape(-1, 1), bm=1024)

    g = y[:N].reshape(N // 420, 420 * C2)
    w4p = jnp.pad(f4W.T, ((0, 0), (0, 118)))
    b4p = jnp.pad(f4b, (0, 118))
    out = _mlp(g, f1W.T, f1b, f2W.T, f2b, f3W.T, f3b, w4p, b4p)
    return out[:, :10].astype(jnp.float64)
